# Initial kernel scaffold; baseline (speedup 1.0000x reference)
import jax
import jax.numpy as jnp
from jax.experimental import pallas as pl


def kernel(xyz, xyz_batch_cnt, new_xyz, new_xyz_batch_cnt, features, params):
    raise NotImplementedError("write your pallas kernel here")


# speedup vs baseline: 22.7387x; 22.7387x over previous
import functools

import jax
import jax.numpy as jnp
from jax import lax
from jax.experimental import pallas as pl
from jax.experimental.pallas import tpu as pltpu
from jax.experimental.pallas import tpu_sc as plsc

_RADII = (0.8, 1.6)
_NS = (16, 32)
_B = 2
_NPTS = 8192
_MQ = 1024
_N = _B * _NPTS
_M = _B * _MQ
_CIN = 32
_EPS = 1e-3
_L = 16
_NW = 32
_QPW = _M // _NW
_NCHUNK = _NPTS // _L


def _sc_ballquery_gather(pxs, pys, pzs, qxs, qys, qzs, features):
    ns0, ns1 = _NS
    r0sq = _RADII[0] * _RADII[0]
    r1sq = _RADII[1] * _RADII[1]
    n0 = _QPW * ns0
    n1 = _QPW * ns1

    mesh = plsc.VectorSubcoreMesh(core_axis_name="c", subcore_axis_name="s")

    out_type = (
        jax.ShapeDtypeStruct((_M * ns0, 4), jnp.float32),
        jax.ShapeDtypeStruct((_M * ns1, 4), jnp.float32),
        jax.ShapeDtypeStruct((_M * ns0, _CIN), jnp.float32),
        jax.ShapeDtypeStruct((_M * ns1, _CIN), jnp.float32),
    )
    scratch_types = [
        pltpu.VMEM((_NPTS,), jnp.float32),
        pltpu.VMEM((_NPTS,), jnp.float32),
        pltpu.VMEM((_NPTS,), jnp.float32),
        pltpu.VMEM((_QPW,), jnp.float32),
        pltpu.VMEM((_QPW,), jnp.float32),
        pltpu.VMEM((_QPW,), jnp.float32),
        pltpu.VMEM((n0,), jnp.int32),
        pltpu.VMEM((n1,), jnp.int32),
        pltpu.VMEM((n0, 4), jnp.float32),
        pltpu.VMEM((n1, 4), jnp.float32),
        pltpu.VMEM((n1, _CIN), jnp.float32),
        pltpu.SemaphoreType.DMA,
    ]

    @functools.partial(pl.kernel, out_type=out_type, mesh=mesh,
                       scratch_types=scratch_types,
                       compiler_params=pltpu.CompilerParams(
                           needs_layout_passes=False,
                           use_tc_tiling_on_sc=False))
    def k(pxs_h, pys_h, pzs_h, qxs_h, qys_h, qzs_h, feat_h,
          gx0_h, gx1_h, gf0_h, gf1_h,
          xs, ys, zs, qx, qy, qz, idx0, idx1, gx0, gx1, rows, sem):
        w = lax.axis_index("c") * 16 + lax.axis_index("s")
        b = w // (_NW // _B)
        pbase = b * _NPTS
        qbase = w * _QPW

        pltpu.sync_copy(pxs_h.at[pl.ds(pbase, _NPTS)], xs)
        pltpu.sync_copy(pys_h.at[pl.ds(pbase, _NPTS)], ys)
        pltpu.sync_copy(pzs_h.at[pl.ds(pbase, _NPTS)], zs)
        pltpu.sync_copy(qxs_h.at[pl.ds(qbase, _QPW)], qx)
        pltpu.sync_copy(qys_h.at[pl.ds(qbase, _QPW)], qy)
        pltpu.sync_copy(qzs_h.at[pl.ds(qbase, _QPW)], qz)

        lane = lax.iota(jnp.int32, _L)
        zeros_i = jnp.zeros((_L,), jnp.int32)
        zeros_f = jnp.zeros((_L,), jnp.float32)

        def per_query(i, carry):
            bi = zeros_i + i
            qxb = plsc.load_gather(qx, [bi])
            qyb = plsc.load_gather(qy, [bi])
            qzb = plsc.load_gather(qz, [bi])
            o0 = i * ns0
            o1 = i * ns1

            def chunk(it, cnts):
                cnt0, cnt1 = cnts
                off = it * _L
                dx = xs[pl.ds(off, _L)] - qxb
                dy = ys[pl.ds(off, _L)] - qyb
                dz = zs[pl.ds(off, _L)] - qzb
                d2 = dx * dx + dy * dy + dz * dz
                gidx = lane + (pbase + off)
                m0 = d2 <= r0sq
                m1 = d2 <= r1sq
                c0 = plsc.cumsum(jnp.where(m0, 1, 0))
                c1 = plsc.cumsum(jnp.where(m1, 1, 0))
                p0 = cnt0 + c0 - 1
                p1 = cnt1 + c1 - 1
                plsc.store_scatter(idx0, [o0 + p0], gidx, mask=m0 & (p0 < ns0))
                plsc.store_scatter(idx1, [o1 + p1], gidx, mask=m1 & (p1 < ns1))
                cnt0 = cnt0 + plsc.all_reduce_population_count(m0)
                cnt1 = cnt1 + plsc.all_reduce_population_count(m1)
                return cnt0, cnt1

            cnt0, cnt1 = lax.fori_loop(0, _NCHUNK, chunk, (zeros_i, zeros_i))

            first0 = plsc.load_gather(idx0, [zeros_i + o0])
            cur0 = idx0[pl.ds(o0, _L)]
            sel0 = jnp.where(lane < cnt0, cur0, first0)
            idx0[pl.ds(o0, _L)] = sel0
            loc0 = sel0 - pbase
            row0 = o0 + lane
            plsc.store_scatter(gx0, [row0, zeros_i], plsc.load_gather(xs, [loc0]) - qxb)
            plsc.store_scatter(gx0, [row0, zeros_i + 1], plsc.load_gather(ys, [loc0]) - qyb)
            plsc.store_scatter(gx0, [row0, zeros_i + 2], plsc.load_gather(zs, [loc0]) - qzb)
            plsc.store_scatter(gx0, [row0, zeros_i + 3], zeros_f)

            first1 = plsc.load_gather(idx1, [zeros_i + o1])
            for h in range(ns1 // _L):
                slot = lane + h * _L
                cur = idx1[pl.ds(o1 + h * _L, _L)]
                sel = jnp.where(slot < cnt1, cur, first1)
                idx1[pl.ds(o1 + h * _L, _L)] = sel
                loc = sel - pbase
                row = o1 + h * _L + lane
                plsc.store_scatter(gx1, [row, zeros_i], plsc.load_gather(xs, [loc]) - qxb)
                plsc.store_scatter(gx1, [row, zeros_i + 1], plsc.load_gather(ys, [loc]) - qyb)
                plsc.store_scatter(gx1, [row, zeros_i + 2], plsc.load_gather(zs, [loc]) - qzb)
                plsc.store_scatter(gx1, [row, zeros_i + 3], zeros_f)
            return carry

        lax.fori_loop(0, _QPW, per_query, 0)

        pltpu.sync_copy(gx0, gx0_h.at[pl.ds(w * n0, n0)])
        pltpu.sync_copy(gx1, gx1_h.at[pl.ds(w * n1, n1)])

        waits = []
        for cs in range(0, n1, 128):
            waits.append(pltpu.async_copy(
                feat_h.at[idx1.at[pl.ds(cs, 128)]], rows.at[pl.ds(cs, 128)], sem))
        for h in waits:
            h.wait()
        pltpu.sync_copy(rows, gf1_h.at[pl.ds(w * n1, n1)])

        waits = []
        for cs in range(0, n0, 128):
            waits.append(pltpu.async_copy(
                feat_h.at[idx0.at[pl.ds(cs, 128)]], rows.at[pl.ds(cs, 128)], sem))
        for h in waits:
            h.wait()
        pltpu.sync_copy(rows.at[pl.ds(0, n0)], gf0_h.at[pl.ds(w * n0, n0)])

    return k(pxs, pys, pzs, qxs, qys, qzs, features)


def _tc_mlp_scale(gx4, gf, layer1, layer2, ns):
    MN = gf.shape[0]
    C1 = layer1["W"].shape[0]
    C2 = layer2["W"].shape[0]
    R = 4096
    NB = MN // R
    qpb = R // ns
    Mq = MN // ns
    cntf = float(MN)

    w1aT = jnp.zeros((4, C1), jnp.float32).at[:3].set(layer1["W"][:, :3].T)
    w1bT = layer1["W"][:, 3:].T
    w2T = layer2["W"].T
    g1 = layer1["g"].reshape(1, C1)
    b1 = layer1["b"].reshape(1, C1)
    g2 = layer2["g"].reshape(1, C2)
    b2 = layer2["b"].reshape(1, C2)

    def body(gx_ref, gf_ref, w1a_ref, w1b_ref, g1_ref, b1_ref, w2_ref,
             z_ref, st2_ref, st1):
        p = pl.program_id(0)
        i = pl.program_id(1)

        @pl.when((p == 0) & (i == 0))
        def _init():
            st1[...] = jnp.zeros_like(st1)
            st2_ref[...] = jnp.zeros_like(st2_ref)

        y1 = (jnp.dot(gf_ref[...], w1b_ref[...], preferred_element_type=jnp.float32)
              + jnp.dot(gx_ref[...], w1a_ref[...], preferred_element_type=jnp.float32))

        @pl.when(p == 0)
        def _pass0():
            st1[0:1, :C1] += jnp.sum(y1, axis=0, keepdims=True)
            st1[1:2, :C1] += jnp.sum(y1 * y1, axis=0, keepdims=True)

        @pl.when(p == 1)
        def _pass1():
            mean1 = st1[0:1, :C1] / cntf
            var1 = st1[1:2, :C1] / cntf - mean1 * mean1
            x = jnp.maximum((y1 - mean1) * (lax.rsqrt(var1 + _EPS) * g1_ref[...])
                            + b1_ref[...], 0.0)
            y2 = jnp.dot(x, w2_ref[...], preferred_element_type=jnp.float32)
            st2_ref[0:1, :C2] += jnp.sum(y2, axis=0, keepdims=True)
            st2_ref[1:2, :C2] += jnp.sum(y2 * y2, axis=0, keepdims=True)
            z_ref[...] = jnp.max(y2.reshape(qpb, ns, C2), axis=1)

    z, st2 = pl.pallas_call(
        body,
        grid=(2, NB),
        in_specs=[
            pl.BlockSpec((R, 4), lambda p, i: (i, 0)),
            pl.BlockSpec((R, _CIN), lambda p, i: (i, 0)),
            pl.BlockSpec((4, C1), lambda p, i: (0, 0)),
            pl.BlockSpec((_CIN, C1), lambda p, i: (0, 0)),
            pl.BlockSpec((1, C1), lambda p, i: (0, 0)),
            pl.BlockSpec((1, C1), lambda p, i: (0, 0)),
            pl.BlockSpec((C1, C2), lambda p, i: (0, 0)),
        ],
        out_specs=[
            pl.BlockSpec((qpb, C2), lambda p, i: (i, 0)),
            pl.BlockSpec((8, 128), lambda p, i: (0, 0)),
        ],
        out_shape=[
            jax.ShapeDtypeStruct((Mq, C2), jnp.float32),
            jax.ShapeDtypeStruct((8, 128), jnp.float32),
        ],
        scratch_shapes=[pltpu.VMEM((8, 128), jnp.float32)],
    )(gx4, gf, w1aT, w1bT, g1, b1, w2T)

    def fin(z_ref, st_ref, g2_ref, b2_ref, out_ref):
        mean = st_ref[0:1, :C2] / cntf
        var = st_ref[1:2, :C2] / cntf - mean * mean
        out_ref[...] = jnp.maximum(
            (z_ref[...] - mean) * (lax.rsqrt(var + _EPS) * g2_ref[...]) + b2_ref[...],
            0.0)

    out = pl.pallas_call(
        fin,
        out_shape=jax.ShapeDtypeStruct((Mq, C2), jnp.float32),
    )(z, st2, g2, b2)
    return out


def kernel(xyz, xyz_batch_cnt, new_xyz, new_xyz_batch_cnt, features, params):
    xyzT = xyz.T
    newT = new_xyz.T
    gx0, gx1, gf0, gf1 = _sc_ballquery_gather(
        xyzT[0], xyzT[1], xyzT[2], newT[0], newT[1], newT[2], features)
    out0 = _tc_mlp_scale(gx0, gf0, params[0][0], params[0][1], _NS[0])
    out1 = _tc_mlp_scale(gx1, gf1, params[1][0], params[1][1], _NS[1])
    new_features = jnp.concatenate([out0, out1], axis=1)
    return new_xyz, new_features


# speedup vs baseline: 32.2640x; 1.4189x over previous
import functools

import jax
import jax.numpy as jnp
from jax import lax
from jax.experimental import pallas as pl
from jax.experimental.pallas import tpu as pltpu
from jax.experimental.pallas import tpu_sc as plsc

_RADII = (0.8, 1.6)
_NS = (16, 32)
_B = 2
_NPTS = 8192
_MQ = 1024
_N = _B * _NPTS
_M = _B * _MQ
_CIN = 32
_EPS = 1e-3
_L = 16
_NW = 32
_QPW = _M // _NW
_NCHUNK = _NPTS // _L


def _sc_ballquery_gather(pxs, pys, pzs, qxs, qys, qzs, features):
    ns0, ns1 = _NS
    r0sq = _RADII[0] * _RADII[0]
    r1sq = _RADII[1] * _RADII[1]
    n0 = _QPW * ns0
    n1 = _QPW * ns1

    mesh = plsc.VectorSubcoreMesh(core_axis_name="c", subcore_axis_name="s")

    out_type = (
        jax.ShapeDtypeStruct((_M * ns0, 4), jnp.float32),
        jax.ShapeDtypeStruct((_M * ns1, 4), jnp.float32),
        jax.ShapeDtypeStruct((_M * ns0, _CIN), jnp.float32),
        jax.ShapeDtypeStruct((_M * ns1, _CIN), jnp.float32),
    )
    scratch_types = [
        pltpu.VMEM((_NPTS,), jnp.float32),
        pltpu.VMEM((_NPTS,), jnp.float32),
        pltpu.VMEM((_NPTS,), jnp.float32),
        pltpu.VMEM((_QPW,), jnp.float32),
        pltpu.VMEM((_QPW,), jnp.float32),
        pltpu.VMEM((_QPW,), jnp.float32),
        pltpu.VMEM((n0,), jnp.int32),
        pltpu.VMEM((n1,), jnp.int32),
        pltpu.VMEM((n0, 4), jnp.float32),
        pltpu.VMEM((n1, 4), jnp.float32),
        pltpu.VMEM((n1, _CIN), jnp.float32),
        pltpu.SemaphoreType.DMA,
    ]

    @functools.partial(pl.kernel, out_type=out_type, mesh=mesh,
                       scratch_types=scratch_types,
                       compiler_params=pltpu.CompilerParams(
                           needs_layout_passes=False,
                           use_tc_tiling_on_sc=False))
    def k(pxs_h, pys_h, pzs_h, qxs_h, qys_h, qzs_h, feat_h,
          gx0_h, gx1_h, gf0_h, gf1_h,
          xs, ys, zs, qx, qy, qz, idx0, idx1, gx0, gx1, rows, sem):
        w = lax.axis_index("c") * 16 + lax.axis_index("s")
        b = w // (_NW // _B)
        pbase = b * _NPTS
        qbase = w * _QPW

        pltpu.sync_copy(pxs_h.at[pl.ds(pbase, _NPTS)], xs)
        pltpu.sync_copy(pys_h.at[pl.ds(pbase, _NPTS)], ys)
        pltpu.sync_copy(pzs_h.at[pl.ds(pbase, _NPTS)], zs)
        pltpu.sync_copy(qxs_h.at[pl.ds(qbase, _QPW)], qx)
        pltpu.sync_copy(qys_h.at[pl.ds(qbase, _QPW)], qy)
        pltpu.sync_copy(qzs_h.at[pl.ds(qbase, _QPW)], qz)

        lane = lax.iota(jnp.int32, _L)
        zeros_i = jnp.zeros((_L,), jnp.int32)
        zeros_f = jnp.zeros((_L,), jnp.float32)

        def per_query(i, carry):
            bi = zeros_i + i
            qxb = plsc.load_gather(qx, [bi])
            qyb = plsc.load_gather(qy, [bi])
            qzb = plsc.load_gather(qz, [bi])
            o0 = i * ns0
            o1 = i * ns1

            def chunk(off, cnt0, cnt1):
                dx = xs[pl.ds(off, _L)] - qxb
                dy = ys[pl.ds(off, _L)] - qyb
                dz = zs[pl.ds(off, _L)] - qzb
                d2 = dx * dx + dy * dy + dz * dz
                gidx = lane + (pbase + off)
                m0 = d2 <= r0sq
                m1 = d2 <= r1sq
                c0 = plsc.cumsum(jnp.where(m0, 1, 0))
                c1 = plsc.cumsum(jnp.where(m1, 1, 0))
                p0 = cnt0 + c0 - 1
                p1 = cnt1 + c1 - 1
                plsc.store_scatter(idx0, [o0 + p0], gidx, mask=m0 & (p0 < ns0))
                plsc.store_scatter(idx1, [o1 + p1], gidx, mask=m1 & (p1 < ns1))
                cnt0 = cnt0 + plsc.all_reduce_population_count(m0)
                cnt1 = cnt1 + plsc.all_reduce_population_count(m1)
                return cnt0, cnt1

            UNROLL = 8
            def scan_cond(st):
                sc_i, cnt0, cnt1 = st
                more = jnp.any((cnt0 < ns0) | (cnt1 < ns1))
                return (sc_i < _NCHUNK // UNROLL) & more

            def scan_body(st):
                sc_i, cnt0, cnt1 = st
                base_off = sc_i * (UNROLL * _L)
                for h in range(UNROLL):
                    cnt0, cnt1 = chunk(base_off + h * _L, cnt0, cnt1)
                return sc_i + 1, cnt0, cnt1

            _, cnt0, cnt1 = lax.while_loop(
                scan_cond, scan_body, (jnp.int32(0), zeros_i, zeros_i))

            first0 = plsc.load_gather(idx0, [zeros_i + o0])
            cur0 = idx0[pl.ds(o0, _L)]
            sel0 = jnp.where(lane < cnt0, cur0, first0)
            idx0[pl.ds(o0, _L)] = sel0
            loc0 = sel0 - pbase
            row0 = o0 + lane
            plsc.store_scatter(gx0, [row0, zeros_i], plsc.load_gather(xs, [loc0]) - qxb)
            plsc.store_scatter(gx0, [row0, zeros_i + 1], plsc.load_gather(ys, [loc0]) - qyb)
            plsc.store_scatter(gx0, [row0, zeros_i + 2], plsc.load_gather(zs, [loc0]) - qzb)
            plsc.store_scatter(gx0, [row0, zeros_i + 3], zeros_f)

            first1 = plsc.load_gather(idx1, [zeros_i + o1])
            for h in range(ns1 // _L):
                slot = lane + h * _L
                cur = idx1[pl.ds(o1 + h * _L, _L)]
                sel = jnp.where(slot < cnt1, cur, first1)
                idx1[pl.ds(o1 + h * _L, _L)] = sel
                loc = sel - pbase
                row = o1 + h * _L + lane
                plsc.store_scatter(gx1, [row, zeros_i], plsc.load_gather(xs, [loc]) - qxb)
                plsc.store_scatter(gx1, [row, zeros_i + 1], plsc.load_gather(ys, [loc]) - qyb)
                plsc.store_scatter(gx1, [row, zeros_i + 2], plsc.load_gather(zs, [loc]) - qzb)
                plsc.store_scatter(gx1, [row, zeros_i + 3], zeros_f)
            return carry

        lax.fori_loop(0, _QPW, per_query, 0)

        pltpu.sync_copy(gx0, gx0_h.at[pl.ds(w * n0, n0)])
        pltpu.sync_copy(gx1, gx1_h.at[pl.ds(w * n1, n1)])

        waits = []
        for cs in range(0, n1, 128):
            waits.append(pltpu.async_copy(
                feat_h.at[idx1.at[pl.ds(cs, 128)]], rows.at[pl.ds(cs, 128)], sem))
        for h in waits:
            h.wait()
        pltpu.sync_copy(rows, gf1_h.at[pl.ds(w * n1, n1)])

        waits = []
        for cs in range(0, n0, 128):
            waits.append(pltpu.async_copy(
                feat_h.at[idx0.at[pl.ds(cs, 128)]], rows.at[pl.ds(cs, 128)], sem))
        for h in waits:
            h.wait()
        pltpu.sync_copy(rows.at[pl.ds(0, n0)], gf0_h.at[pl.ds(w * n0, n0)])

    return k(pxs, pys, pzs, qxs, qys, qzs, features)


def _tc_mlp_scale(gx4, gf, layer1, layer2, ns):
    MN = gf.shape[0]
    C1 = layer1["W"].shape[0]
    C2 = layer2["W"].shape[0]
    R = 4096
    NB = MN // R
    qpb = R // ns
    Mq = MN // ns
    cntf = float(MN)

    w1aT = jnp.zeros((4, C1), jnp.float32).at[:3].set(layer1["W"][:, :3].T)
    w1bT = layer1["W"][:, 3:].T
    w2T = layer2["W"].T
    g1 = layer1["g"].reshape(1, C1)
    b1 = layer1["b"].reshape(1, C1)
    g2 = layer2["g"].reshape(1, C2)
    b2 = layer2["b"].reshape(1, C2)

    def body(gx_ref, gf_ref, w1a_ref, w1b_ref, g1_ref, b1_ref, w2_ref,
             z_ref, st2_ref, st1):
        p = pl.program_id(0)
        i = pl.program_id(1)

        @pl.when((p == 0) & (i == 0))
        def _init():
            st1[...] = jnp.zeros_like(st1)
            st2_ref[...] = jnp.zeros_like(st2_ref)

        y1 = (jnp.dot(gf_ref[...], w1b_ref[...], preferred_element_type=jnp.float32)
              + jnp.dot(gx_ref[...], w1a_ref[...], preferred_element_type=jnp.float32))

        @pl.when(p == 0)
        def _pass0():
            st1[0:1, :C1] += jnp.sum(y1, axis=0, keepdims=True)
            st1[1:2, :C1] += jnp.sum(y1 * y1, axis=0, keepdims=True)

        @pl.when(p == 1)
        def _pass1():
            mean1 = st1[0:1, :C1] / cntf
            var1 = st1[1:2, :C1] / cntf - mean1 * mean1
            x = jnp.maximum((y1 - mean1) * (lax.rsqrt(var1 + _EPS) * g1_ref[...])
                            + b1_ref[...], 0.0)
            y2 = jnp.dot(x, w2_ref[...], preferred_element_type=jnp.float32)
            st2_ref[0:1, :C2] += jnp.sum(y2, axis=0, keepdims=True)
            st2_ref[1:2, :C2] += jnp.sum(y2 * y2, axis=0, keepdims=True)
            z_ref[...] = jnp.max(y2.reshape(qpb, ns, C2), axis=1)

    z, st2 = pl.pallas_call(
        body,
        grid=(2, NB),
        in_specs=[
            pl.BlockSpec((R, 4), lambda p, i: (i, 0)),
            pl.BlockSpec((R, _CIN), lambda p, i: (i, 0)),
            pl.BlockSpec((4, C1), lambda p, i: (0, 0)),
            pl.BlockSpec((_CIN, C1), lambda p, i: (0, 0)),
            pl.BlockSpec((1, C1), lambda p, i: (0, 0)),
            pl.BlockSpec((1, C1), lambda p, i: (0, 0)),
            pl.BlockSpec((C1, C2), lambda p, i: (0, 0)),
        ],
        out_specs=[
            pl.BlockSpec((qpb, C2), lambda p, i: (i, 0)),
            pl.BlockSpec((8, 128), lambda p, i: (0, 0)),
        ],
        out_shape=[
            jax.ShapeDtypeStruct((Mq, C2), jnp.float32),
            jax.ShapeDtypeStruct((8, 128), jnp.float32),
        ],
        scratch_shapes=[pltpu.VMEM((8, 128), jnp.float32)],
    )(gx4, gf, w1aT, w1bT, g1, b1, w2T)

    def fin(z_ref, st_ref, g2_ref, b2_ref, out_ref):
        mean = st_ref[0:1, :C2] / cntf
        var = st_ref[1:2, :C2] / cntf - mean * mean
        out_ref[...] = jnp.maximum(
            (z_ref[...] - mean) * (lax.rsqrt(var + _EPS) * g2_ref[...]) + b2_ref[...],
            0.0)

    out = pl.pallas_call(
        fin,
        out_shape=jax.ShapeDtypeStruct((Mq, C2), jnp.float32),
    )(z, st2, g2, b2)
    return out


def kernel(xyz, xyz_batch_cnt, new_xyz, new_xyz_batch_cnt, features, params):
    xyzT = xyz.T
    newT = new_xyz.T
    gx0, gx1, gf0, gf1 = _sc_ballquery_gather(
        xyzT[0], xyzT[1], xyzT[2], newT[0], newT[1], newT[2], features)
    out0 = _tc_mlp_scale(gx0, gf0, params[0][0], params[0][1], _NS[0])
    out1 = _tc_mlp_scale(gx1, gf1, params[1][0], params[1][1], _NS[1])
    new_features = jnp.concatenate([out0, out1], axis=1)
    return new_xyz, new_features


# speedup vs baseline: 56.2084x; 1.7421x over previous
import functools

import jax
import jax.numpy as jnp
from jax import lax
from jax.experimental import pallas as pl
from jax.experimental.pallas import tpu as pltpu
from jax.experimental.pallas import tpu_sc as plsc

_RADII = (0.8, 1.6)
_NS = (16, 32)
_B = 2
_NPTS = 8192
_MQ = 1024
_N = _B * _NPTS
_M = _B * _MQ
_CIN = 32
_EPS = 1e-3
_L = 16
_NW = 32
_QPW = _M // _NW
_NCHUNK = _NPTS // _L


def _sc_ballquery_gather(pxs, pys, pzs, qxs, qys, qzs, features):
    ns0, ns1 = _NS
    r0sq = _RADII[0] * _RADII[0]
    r1sq = _RADII[1] * _RADII[1]
    n0 = _QPW * ns0
    n1 = _QPW * ns1

    mesh = plsc.VectorSubcoreMesh(core_axis_name="c", subcore_axis_name="s")

    out_type = (
        jax.ShapeDtypeStruct((_M * ns0, 4), jnp.float32),
        jax.ShapeDtypeStruct((_M * ns1, 4), jnp.float32),
        jax.ShapeDtypeStruct((_M * ns0, _CIN), jnp.float32),
        jax.ShapeDtypeStruct((_M * ns1, _CIN), jnp.float32),
    )
    scratch_types = [
        pltpu.VMEM((_NPTS,), jnp.float32),
        pltpu.VMEM((_NPTS,), jnp.float32),
        pltpu.VMEM((_NPTS,), jnp.float32),
        pltpu.VMEM((_QPW,), jnp.float32),
        pltpu.VMEM((_QPW,), jnp.float32),
        pltpu.VMEM((_QPW,), jnp.float32),
        pltpu.VMEM((n0,), jnp.int32),
        pltpu.VMEM((n1,), jnp.int32),
        pltpu.VMEM((n0, 4), jnp.float32),
        pltpu.VMEM((n1, 4), jnp.float32),
        pltpu.VMEM((n1, _CIN), jnp.float32),
        pltpu.SemaphoreType.DMA,
    ]

    @functools.partial(pl.kernel, out_type=out_type, mesh=mesh,
                       scratch_types=scratch_types,
                       compiler_params=pltpu.CompilerParams(
                           needs_layout_passes=False,
                           use_tc_tiling_on_sc=False))
    def k(pxs_h, pys_h, pzs_h, qxs_h, qys_h, qzs_h, feat_h,
          gx0_h, gx1_h, gf0_h, gf1_h,
          xs, ys, zs, qx, qy, qz, idx0, idx1, gx0, gx1, rows, sem):
        w = lax.axis_index("c") * 16 + lax.axis_index("s")
        b = w // (_NW // _B)
        pbase = b * _NPTS
        qbase = w * _QPW

        pltpu.sync_copy(pxs_h.at[pl.ds(pbase, _NPTS)], xs)
        pltpu.sync_copy(pys_h.at[pl.ds(pbase, _NPTS)], ys)
        pltpu.sync_copy(pzs_h.at[pl.ds(pbase, _NPTS)], zs)
        pltpu.sync_copy(qxs_h.at[pl.ds(qbase, _QPW)], qx)
        pltpu.sync_copy(qys_h.at[pl.ds(qbase, _QPW)], qy)
        pltpu.sync_copy(qzs_h.at[pl.ds(qbase, _QPW)], qz)

        lane = lax.iota(jnp.int32, _L)
        zeros_i = jnp.zeros((_L,), jnp.int32)
        zeros_f = jnp.zeros((_L,), jnp.float32)

        def per_query(i, carry):
            bi = zeros_i + i
            qxb = plsc.load_gather(qx, [bi])
            qyb = plsc.load_gather(qy, [bi])
            qzb = plsc.load_gather(qz, [bi])
            o0 = i * ns0
            o1 = i * ns1

            def chunk(off, cnt0, cnt1):
                dx = xs[pl.ds(off, _L)] - qxb
                dy = ys[pl.ds(off, _L)] - qyb
                dz = zs[pl.ds(off, _L)] - qzb
                d2 = dx * dx + dy * dy + dz * dz
                gidx = lane + (pbase + off)
                m0 = d2 <= r0sq
                m1 = d2 <= r1sq
                c0 = plsc.cumsum(jnp.where(m0, 1, 0))
                c1 = plsc.cumsum(jnp.where(m1, 1, 0))
                p0 = cnt0 + c0 - 1
                p1 = cnt1 + c1 - 1
                plsc.store_scatter(idx0, [o0 + p0], gidx, mask=m0 & (p0 < ns0))
                plsc.store_scatter(idx1, [o1 + p1], gidx, mask=m1 & (p1 < ns1))
                cnt0 = cnt0 + plsc.all_reduce_population_count(m0)
                cnt1 = cnt1 + plsc.all_reduce_population_count(m1)
                return cnt0, cnt1

            UNROLL = 8
            def scan_cond(st):
                sc_i, cnt0, cnt1 = st
                more = jnp.any((cnt0 < ns0) | (cnt1 < ns1))
                return (sc_i < _NCHUNK // UNROLL) & more

            def scan_body(st):
                sc_i, cnt0, cnt1 = st
                base_off = sc_i * (UNROLL * _L)
                cnt0, cnt1 = plsc.parallel_loop(
                    base_off, base_off + UNROLL * _L, _L, unroll=UNROLL,
                    carry=(cnt0, cnt1))(
                        lambda off, c: chunk(off, c[0], c[1]))
                return sc_i + 1, cnt0, cnt1

            _, cnt0, cnt1 = lax.while_loop(
                scan_cond, scan_body, (jnp.int32(0), zeros_i, zeros_i))

            first0 = plsc.load_gather(idx0, [zeros_i + o0])
            cur0 = idx0[pl.ds(o0, _L)]
            sel0 = jnp.where(lane < cnt0, cur0, first0)
            idx0[pl.ds(o0, _L)] = sel0
            loc0 = sel0 - pbase
            row0 = o0 + lane
            plsc.store_scatter(gx0, [row0, zeros_i], plsc.load_gather(xs, [loc0]) - qxb)
            plsc.store_scatter(gx0, [row0, zeros_i + 1], plsc.load_gather(ys, [loc0]) - qyb)
            plsc.store_scatter(gx0, [row0, zeros_i + 2], plsc.load_gather(zs, [loc0]) - qzb)
            plsc.store_scatter(gx0, [row0, zeros_i + 3], zeros_f)

            first1 = plsc.load_gather(idx1, [zeros_i + o1])
            for h in range(ns1 // _L):
                slot = lane + h * _L
                cur = idx1[pl.ds(o1 + h * _L, _L)]
                sel = jnp.where(slot < cnt1, cur, first1)
                idx1[pl.ds(o1 + h * _L, _L)] = sel
                loc = sel - pbase
                row = o1 + h * _L + lane
                plsc.store_scatter(gx1, [row, zeros_i], plsc.load_gather(xs, [loc]) - qxb)
                plsc.store_scatter(gx1, [row, zeros_i + 1], plsc.load_gather(ys, [loc]) - qyb)
                plsc.store_scatter(gx1, [row, zeros_i + 2], plsc.load_gather(zs, [loc]) - qzb)
                plsc.store_scatter(gx1, [row, zeros_i + 3], zeros_f)
            return carry

        lax.fori_loop(0, _QPW, per_query, 0)

        pltpu.sync_copy(gx0, gx0_h.at[pl.ds(w * n0, n0)])
        pltpu.sync_copy(gx1, gx1_h.at[pl.ds(w * n1, n1)])

        waits = []
        for cs in range(0, n1, 128):
            waits.append(pltpu.async_copy(
                feat_h.at[idx1.at[pl.ds(cs, 128)]], rows.at[pl.ds(cs, 128)], sem))
        for h in waits:
            h.wait()
        pltpu.sync_copy(rows, gf1_h.at[pl.ds(w * n1, n1)])

        waits = []
        for cs in range(0, n0, 128):
            waits.append(pltpu.async_copy(
                feat_h.at[idx0.at[pl.ds(cs, 128)]], rows.at[pl.ds(cs, 128)], sem))
        for h in waits:
            h.wait()
        pltpu.sync_copy(rows.at[pl.ds(0, n0)], gf0_h.at[pl.ds(w * n0, n0)])

    return k(pxs, pys, pzs, qxs, qys, qzs, features)


def _tc_mlp_scale(gx4, gf, layer1, layer2, ns):
    MN = gf.shape[0]
    C1 = layer1["W"].shape[0]
    C2 = layer2["W"].shape[0]
    R = 4096
    NB = MN // R
    qpb = R // ns
    Mq = MN // ns
    cntf = float(MN)

    w1aT = jnp.zeros((4, C1), jnp.float32).at[:3].set(layer1["W"][:, :3].T)
    w1bT = layer1["W"][:, 3:].T
    w2T = layer2["W"].T
    g1 = layer1["g"].reshape(1, C1)
    b1 = layer1["b"].reshape(1, C1)
    g2 = layer2["g"].reshape(1, C2)
    b2 = layer2["b"].reshape(1, C2)

    def body(gx_ref, gf_ref, w1a_ref, w1b_ref, g1_ref, b1_ref, w2_ref,
             z_ref, st2_ref, st1):
        p = pl.program_id(0)
        i = pl.program_id(1)

        @pl.when((p == 0) & (i == 0))
        def _init():
            st1[...] = jnp.zeros_like(st1)
            st2_ref[...] = jnp.zeros_like(st2_ref)

        y1 = (jnp.dot(gf_ref[...], w1b_ref[...], preferred_element_type=jnp.float32)
              + jnp.dot(gx_ref[...], w1a_ref[...], preferred_element_type=jnp.float32))

        @pl.when(p == 0)
        def _pass0():
            st1[0:1, :C1] += jnp.sum(y1, axis=0, keepdims=True)
            st1[1:2, :C1] += jnp.sum(y1 * y1, axis=0, keepdims=True)

        @pl.when(p == 1)
        def _pass1():
            mean1 = st1[0:1, :C1] / cntf
            var1 = st1[1:2, :C1] / cntf - mean1 * mean1
            x = jnp.maximum((y1 - mean1) * (lax.rsqrt(var1 + _EPS) * g1_ref[...])
                            + b1_ref[...], 0.0)
            y2 = jnp.dot(x, w2_ref[...], preferred_element_type=jnp.float32)
            st2_ref[0:1, :C2] += jnp.sum(y2, axis=0, keepdims=True)
            st2_ref[1:2, :C2] += jnp.sum(y2 * y2, axis=0, keepdims=True)
            z_ref[...] = jnp.max(y2.reshape(qpb, ns, C2), axis=1)

    z, st2 = pl.pallas_call(
        body,
        grid=(2, NB),
        in_specs=[
            pl.BlockSpec((R, 4), lambda p, i: (i, 0)),
            pl.BlockSpec((R, _CIN), lambda p, i: (i, 0)),
            pl.BlockSpec((4, C1), lambda p, i: (0, 0)),
            pl.BlockSpec((_CIN, C1), lambda p, i: (0, 0)),
            pl.BlockSpec((1, C1), lambda p, i: (0, 0)),
            pl.BlockSpec((1, C1), lambda p, i: (0, 0)),
            pl.BlockSpec((C1, C2), lambda p, i: (0, 0)),
        ],
        out_specs=[
            pl.BlockSpec((qpb, C2), lambda p, i: (i, 0)),
            pl.BlockSpec((8, 128), lambda p, i: (0, 0)),
        ],
        out_shape=[
            jax.ShapeDtypeStruct((Mq, C2), jnp.float32),
            jax.ShapeDtypeStruct((8, 128), jnp.float32),
        ],
        scratch_shapes=[pltpu.VMEM((8, 128), jnp.float32)],
    )(gx4, gf, w1aT, w1bT, g1, b1, w2T)

    def fin(z_ref, st_ref, g2_ref, b2_ref, out_ref):
        mean = st_ref[0:1, :C2] / cntf
        var = st_ref[1:2, :C2] / cntf - mean * mean
        out_ref[...] = jnp.maximum(
            (z_ref[...] - mean) * (lax.rsqrt(var + _EPS) * g2_ref[...]) + b2_ref[...],
            0.0)

    out = pl.pallas_call(
        fin,
        out_shape=jax.ShapeDtypeStruct((Mq, C2), jnp.float32),
    )(z, st2, g2, b2)
    return out


def kernel(xyz, xyz_batch_cnt, new_xyz, new_xyz_batch_cnt, features, params):
    xyzT = xyz.T
    newT = new_xyz.T
    gx0, gx1, gf0, gf1 = _sc_ballquery_gather(
        xyzT[0], xyzT[1], xyzT[2], newT[0], newT[1], newT[2], features)
    out0 = _tc_mlp_scale(gx0, gf0, params[0][0], params[0][1], _NS[0])
    out1 = _tc_mlp_scale(gx1, gf1, params[1][0], params[1][1], _NS[1])
    new_features = jnp.concatenate([out0, out1], axis=1)
    return new_xyz, new_features


# speedup vs baseline: 58.2895x; 1.0370x over previous
import functools

import jax
import jax.numpy as jnp
from jax import lax
from jax.experimental import pallas as pl
from jax.experimental.pallas import tpu as pltpu
from jax.experimental.pallas import tpu_sc as plsc

_RADII = (0.8, 1.6)
_NS = (16, 32)
_B = 2
_NPTS = 8192
_MQ = 1024
_N = _B * _NPTS
_M = _B * _MQ
_CIN = 32
_EPS = 1e-3
_L = 16
_NW = 32
_QPW = _M // _NW
_NCHUNK = _NPTS // _L
_UNROLL = 8


def _sc_ballquery_gather(xyzf, newf, features):
    ns0, ns1 = _NS
    r0sq = _RADII[0] * _RADII[0]
    r1sq = _RADII[1] * _RADII[1]
    n0 = _QPW * ns0
    n1 = _QPW * ns1

    mesh = plsc.VectorSubcoreMesh(core_axis_name="c", subcore_axis_name="s")

    out_type = (
        jax.ShapeDtypeStruct((_M * ns0, 4), jnp.float32),
        jax.ShapeDtypeStruct((_M * ns1, 4), jnp.float32),
        jax.ShapeDtypeStruct((_M * ns0, _CIN), jnp.float32),
        jax.ShapeDtypeStruct((_M * ns1, _CIN), jnp.float32),
    )
    scratch_types = [
        pltpu.VMEM((_NPTS * 3,), jnp.float32),
        pltpu.VMEM((_NPTS,), jnp.float32),
        pltpu.VMEM((_NPTS,), jnp.float32),
        pltpu.VMEM((_NPTS,), jnp.float32),
        pltpu.VMEM((_QPW * 3,), jnp.float32),
        pltpu.VMEM((n0,), jnp.int32),
        pltpu.VMEM((n1,), jnp.int32),
        pltpu.VMEM((n0, 4), jnp.float32),
        pltpu.VMEM((n1, 4), jnp.float32),
        pltpu.VMEM((1024, _CIN), jnp.float32),
        pltpu.SemaphoreType.DMA,
    ]

    @functools.partial(pl.kernel, out_type=out_type, mesh=mesh,
                       scratch_types=scratch_types,
                       compiler_params=pltpu.CompilerParams(
                           needs_layout_passes=False,
                           use_tc_tiling_on_sc=False))
    def k(xyzf_h, newf_h, feat_h, gx0_h, gx1_h, gf0_h, gf1_h,
          pf, xs, ys, zs, qf, idx0, idx1, gx0, gx1, rows, sem):
        w = lax.axis_index("c") * 16 + lax.axis_index("s")
        b = w // (_NW // _B)
        pbase = b * _NPTS
        qbase = w * _QPW

        pltpu.sync_copy(xyzf_h.at[pl.ds(pbase * 3, _NPTS * 3)], pf)
        pltpu.sync_copy(newf_h.at[pl.ds(qbase * 3, _QPW * 3)], qf)

        lane = lax.iota(jnp.int32, _L)
        lane3 = lane * 3
        zeros_i = jnp.zeros((_L,), jnp.int32)
        zeros_f = jnp.zeros((_L,), jnp.float32)

        def soa(it, _):
            base = it * _L
            src = lane3 + base * 3
            xs[pl.ds(base, _L)] = plsc.load_gather(pf, [src])
            ys[pl.ds(base, _L)] = plsc.load_gather(pf, [src + 1])
            zs[pl.ds(base, _L)] = plsc.load_gather(pf, [src + 2])
            return 0
        plsc.parallel_loop(0, _NCHUNK, 1, unroll=8, carry=jnp.int32(0))(soa)

        def per_query(i, carry):
            q3 = zeros_i + i * 3
            qxb = plsc.load_gather(qf, [q3])
            qyb = plsc.load_gather(qf, [q3 + 1])
            qzb = plsc.load_gather(qf, [q3 + 2])
            o0 = i * ns0
            o1 = i * ns1
            cnt0_init = zeros_i + (o0 - 1)
            cnt1_init = zeros_i + (o1 - 1)
            lim0 = zeros_i + (o0 + ns0)
            lim1 = zeros_i + (o1 + ns1)

            def chunk(off, cnt0, cnt1):
                dx = xs[pl.ds(off, _L)] - qxb
                dy = ys[pl.ds(off, _L)] - qyb
                dz = zs[pl.ds(off, _L)] - qzb
                d2 = dx * dx + dy * dy + dz * dz
                gidx = lane + (pbase + off)
                m0 = d2 <= r0sq
                m1 = d2 <= r1sq
                c0 = plsc.cumsum(jnp.where(m0, 1, 0))
                c1 = plsc.cumsum(jnp.where(m1, 1, 0))
                p0 = cnt0 + c0
                p1 = cnt1 + c1
                plsc.store_scatter(idx0, [p0], gidx, mask=m0 & (p0 < lim0))
                plsc.store_scatter(idx1, [p1], gidx, mask=m1 & (p1 < lim1))
                cnt0 = cnt0 + plsc.all_reduce_population_count(m0)
                cnt1 = cnt1 + plsc.all_reduce_population_count(m1)
                return cnt0, cnt1

            def scan_cond(st):
                sc_i, cnt0, cnt1 = st
                more = jnp.any((cnt0 < lim0 - 1) | (cnt1 < lim1 - 1))
                return (sc_i < _NCHUNK // _UNROLL) & more

            def scan_body(st):
                sc_i, cnt0, cnt1 = st
                base_off = sc_i * (_UNROLL * _L)
                cnt0, cnt1 = plsc.parallel_loop(
                    base_off, base_off + _UNROLL * _L, _L, unroll=_UNROLL,
                    carry=(cnt0, cnt1))(
                        lambda off, c: chunk(off, c[0], c[1]))
                return sc_i + 1, cnt0, cnt1

            _, cnt0, cnt1 = lax.while_loop(
                scan_cond, scan_body, (jnp.int32(0), cnt0_init, cnt1_init))

            laneo0 = lane + o0
            laneo1 = lane + o1
            first0 = plsc.load_gather(idx0, [zeros_i + o0])
            cur0 = idx0[pl.ds(o0, _L)]
            sel0 = jnp.where(laneo0 - 1 < cnt0, cur0, first0)
            idx0[pl.ds(o0, _L)] = sel0
            loc0 = (sel0 - pbase) * 3
            row0 = laneo0
            plsc.store_scatter(gx0, [row0, zeros_i], plsc.load_gather(pf, [loc0]) - qxb)
            plsc.store_scatter(gx0, [row0, zeros_i + 1], plsc.load_gather(pf, [loc0 + 1]) - qyb)
            plsc.store_scatter(gx0, [row0, zeros_i + 2], plsc.load_gather(pf, [loc0 + 2]) - qzb)
            plsc.store_scatter(gx0, [row0, zeros_i + 3], zeros_f)

            first1 = plsc.load_gather(idx1, [zeros_i + o1])
            for h in range(ns1 // _L):
                cur = idx1[pl.ds(o1 + h * _L, _L)]
                sel = jnp.where(laneo1 + (h * _L - 1) < cnt1, cur, first1)
                idx1[pl.ds(o1 + h * _L, _L)] = sel
                loc = (sel - pbase) * 3
                row = laneo1 + h * _L
                plsc.store_scatter(gx1, [row, zeros_i], plsc.load_gather(pf, [loc]) - qxb)
                plsc.store_scatter(gx1, [row, zeros_i + 1], plsc.load_gather(pf, [loc + 1]) - qyb)
                plsc.store_scatter(gx1, [row, zeros_i + 2], plsc.load_gather(pf, [loc + 2]) - qzb)
                plsc.store_scatter(gx1, [row, zeros_i + 3], zeros_f)
            return carry

        lax.fori_loop(0, _QPW, per_query, 0)

        pltpu.sync_copy(gx0, gx0_h.at[pl.ds(w * n0, n0)])
        pltpu.sync_copy(gx1, gx1_h.at[pl.ds(w * n1, n1)])

        for half in range(2):
            hb = half * 1024
            waits = []
            for cs in range(0, 1024, 128):
                waits.append(pltpu.async_copy(
                    feat_h.at[idx1.at[pl.ds(hb + cs, 128)]],
                    rows.at[pl.ds(cs, 128)], sem))
            for hh in waits:
                hh.wait()
            pltpu.sync_copy(rows, gf1_h.at[pl.ds(w * n1 + hb, 1024)])

        waits = []
        for cs in range(0, n0, 128):
            waits.append(pltpu.async_copy(
                feat_h.at[idx0.at[pl.ds(cs, 128)]], rows.at[pl.ds(cs, 128)], sem))
        for hh in waits:
            hh.wait()
        pltpu.sync_copy(rows, gf0_h.at[pl.ds(w * n0, n0)])

    return k(xyzf, newf, features)


def _tc_mlp_scale(gx4, gf, layer1, layer2, ns):
    MN = gf.shape[0]
    C1 = layer1["W"].shape[0]
    C2 = layer2["W"].shape[0]
    R = 8192
    NB = MN // R
    qpb = R // ns
    Mq = MN // ns
    cntf = float(MN)

    w1aT = jnp.zeros((4, C1), jnp.float32).at[:3].set(
        layer1["W"][:, :3].T).astype(jnp.bfloat16)
    w1bT = layer1["W"][:, 3:].T.astype(jnp.bfloat16)
    w2T = layer2["W"].T.astype(jnp.bfloat16)
    g1 = layer1["g"].reshape(1, C1)
    b1 = layer1["b"].reshape(1, C1)
    g2 = layer2["g"].reshape(1, C2)
    b2 = layer2["b"].reshape(1, C2)

    def body(gx_ref, gf_ref, w1a_ref, w1b_ref, g1_ref, b1_ref, w2_ref,
             z_ref, st2_ref, st1):
        p = pl.program_id(0)
        i = pl.program_id(1)

        @pl.when((p == 0) & (i == 0))
        def _init():
            st1[...] = jnp.zeros_like(st1)
            st2_ref[...] = jnp.zeros_like(st2_ref)

        y1 = (jnp.dot(gf_ref[...].astype(jnp.bfloat16), w1b_ref[...],
                      preferred_element_type=jnp.float32)
              + jnp.dot(gx_ref[...].astype(jnp.bfloat16), w1a_ref[...],
                        preferred_element_type=jnp.float32))

        @pl.when(p == 0)
        def _pass0():
            st1[0:1, :C1] += jnp.sum(y1, axis=0, keepdims=True)
            st1[1:2, :C1] += jnp.sum(y1 * y1, axis=0, keepdims=True)

        @pl.when(p == 1)
        def _pass1():
            mean1 = st1[0:1, :C1] / cntf
            var1 = st1[1:2, :C1] / cntf - mean1 * mean1
            x = jnp.maximum((y1 - mean1) * (lax.rsqrt(var1 + _EPS) * g1_ref[...])
                            + b1_ref[...], 0.0)
            y2 = jnp.dot(x.astype(jnp.bfloat16), w2_ref[...],
                         preferred_element_type=jnp.float32)
            st2_ref[0:1, :C2] += jnp.sum(y2, axis=0, keepdims=True)
            st2_ref[1:2, :C2] += jnp.sum(y2 * y2, axis=0, keepdims=True)
            z_ref[...] = jnp.max(y2.reshape(qpb, ns, C2), axis=1)

    z, st2 = pl.pallas_call(
        body,
        grid=(2, NB),
        in_specs=[
            pl.BlockSpec((R, 4), lambda p, i: (i, 0)),
            pl.BlockSpec((R, _CIN), lambda p, i: (i, 0)),
            pl.BlockSpec((4, C1), lambda p, i: (0, 0)),
            pl.BlockSpec((_CIN, C1), lambda p, i: (0, 0)),
            pl.BlockSpec((1, C1), lambda p, i: (0, 0)),
            pl.BlockSpec((1, C1), lambda p, i: (0, 0)),
            pl.BlockSpec((C1, C2), lambda p, i: (0, 0)),
        ],
        out_specs=[
            pl.BlockSpec((qpb, C2), lambda p, i: (i, 0)),
            pl.BlockSpec((8, 128), lambda p, i: (0, 0)),
        ],
        out_shape=[
            jax.ShapeDtypeStruct((Mq, C2), jnp.float32),
            jax.ShapeDtypeStruct((8, 128), jnp.float32),
        ],
        scratch_shapes=[pltpu.VMEM((8, 128), jnp.float32)],
    )(gx4, gf, w1aT, w1bT, g1, b1, w2T)

    def fin(z_ref, st_ref, g2_ref, b2_ref, out_ref):
        mean = st_ref[0:1, :C2] / cntf
        var = st_ref[1:2, :C2] / cntf - mean * mean
        out_ref[...] = jnp.maximum(
            (z_ref[...] - mean) * (lax.rsqrt(var + _EPS) * g2_ref[...]) + b2_ref[...],
            0.0)

    out = pl.pallas_call(
        fin,
        out_shape=jax.ShapeDtypeStruct((Mq, C2), jnp.float32),
    )(z, st2, g2, b2)
    return out


def kernel(xyz, xyz_batch_cnt, new_xyz, new_xyz_batch_cnt, features, params):
    gx0, gx1, gf0, gf1 = _sc_ballquery_gather(
        xyz.reshape(-1), new_xyz.reshape(-1), features)
    out0 = _tc_mlp_scale(gx0, gf0, params[0][0], params[0][1], _NS[0])
    out1 = _tc_mlp_scale(gx1, gf1, params[1][0], params[1][1], _NS[1])
    new_features = jnp.concatenate([out0, out1], axis=1)
    return new_xyz, new_features


# speedup vs baseline: 60.0197x; 1.0297x over previous
import functools

import jax
import jax.numpy as jnp
from jax import lax
from jax.experimental import pallas as pl
from jax.experimental.pallas import tpu as pltpu
from jax.experimental.pallas import tpu_sc as plsc

_RADII = (0.8, 1.6)
_NS = (16, 32)
_B = 2
_NPTS = 8192
_MQ = 1024
_N = _B * _NPTS
_M = _B * _MQ
_CIN = 32
_EPS = 1e-3
_L = 16
_NW = 32
_QPW = _M // _NW
_NCHUNK = _NPTS // _L
_UNROLL = 8


def _sc_ballquery_gather(xyz, new_xyz, features):
    ns0, ns1 = _NS
    r0sq = _RADII[0] * _RADII[0]
    r1sq = _RADII[1] * _RADII[1]
    n0 = _QPW * ns0
    n1 = _QPW * ns1

    mesh = plsc.VectorSubcoreMesh(core_axis_name="c", subcore_axis_name="s")

    out_type = (
        jax.ShapeDtypeStruct((_M * ns0, 4), jnp.float32),
        jax.ShapeDtypeStruct((_M * ns1, 4), jnp.float32),
        jax.ShapeDtypeStruct((_M * ns0, _CIN), jnp.float32),
        jax.ShapeDtypeStruct((_M * ns1, _CIN), jnp.float32),
    )
    scratch_types = [
        pltpu.VMEM((_NPTS * 3,), jnp.float32),
        pltpu.VMEM((_NPTS,), jnp.float32),
        pltpu.VMEM((_NPTS,), jnp.float32),
        pltpu.VMEM((_NPTS,), jnp.float32),
        pltpu.VMEM((_QPW * 3,), jnp.float32),
        pltpu.VMEM((n0,), jnp.int32),
        pltpu.VMEM((n1,), jnp.int32),
        pltpu.VMEM((n0, 4), jnp.float32),
        pltpu.VMEM((n1, 4), jnp.float32),
        pltpu.VMEM((1024, _CIN), jnp.float32),
        pltpu.SemaphoreType.DMA,
    ]

    @functools.partial(pl.kernel, out_type=out_type, mesh=mesh,
                       scratch_types=scratch_types,
                       compiler_params=pltpu.CompilerParams(
                           needs_layout_passes=False,
                           use_tc_tiling_on_sc=False))
    def k(xyz_h, new_h, feat_h, gx0_h, gx1_h, gf0_h, gf1_h,
          pf, xs, ys, zs, qf, idx0, idx1, gx0, gx1, rows, sem):
        w = lax.axis_index("c") * 16 + lax.axis_index("s")
        b = w // (_NW // _B)
        pbase = b * _NPTS
        qbase = w * _QPW

        pltpu.sync_copy(xyz_h.at[pl.ds(pbase * 3, _NPTS * 3)], pf)
        pltpu.sync_copy(new_h.at[pl.ds(qbase * 3, _QPW * 3)], qf)

        lane = lax.iota(jnp.int32, _L)
        lane3 = lane * 3
        zeros_i = jnp.zeros((_L,), jnp.int32)
        zeros_f = jnp.zeros((_L,), jnp.float32)

        def soa(it, _):
            base = it * _L
            src = lane3 + base * 3
            xs[pl.ds(base, _L)] = plsc.load_gather(pf, [src])
            ys[pl.ds(base, _L)] = plsc.load_gather(pf, [src + 1])
            zs[pl.ds(base, _L)] = plsc.load_gather(pf, [src + 2])
            return 0
        plsc.parallel_loop(0, _NCHUNK, 1, unroll=8, carry=jnp.int32(0))(soa)

        def per_query(i, carry):
            q3 = zeros_i + i * 3
            qxb = plsc.load_gather(qf, [q3])
            qyb = plsc.load_gather(qf, [q3 + 1])
            qzb = plsc.load_gather(qf, [q3 + 2])
            o0 = i * ns0
            o1 = i * ns1
            cnt0_init = zeros_i + (o0 - 1)
            cnt1_init = zeros_i + (o1 - 1)
            lim0 = zeros_i + (o0 + ns0)
            lim1 = zeros_i + (o1 + ns1)

            def dists(off):
                dx = xs[pl.ds(off, _L)] - qxb
                dy = ys[pl.ds(off, _L)] - qyb
                dz = zs[pl.ds(off, _L)] - qzb
                return dx * dx + dy * dy + dz * dz

            def chunk_both(off, cnt0, cnt1):
                d2 = dists(off)
                gidx = lane + (pbase + off)
                m0 = d2 <= r0sq
                m1 = d2 <= r1sq
                c0 = plsc.cumsum(jnp.where(m0, 1, 0))
                c1 = plsc.cumsum(jnp.where(m1, 1, 0))
                p0 = cnt0 + c0
                p1 = cnt1 + c1
                plsc.store_scatter(idx0, [p0], gidx, mask=m0 & (p0 < lim0))
                plsc.store_scatter(idx1, [p1], gidx, mask=m1 & (p1 < lim1))
                cnt0 = cnt0 + plsc.all_reduce_population_count(m0)
                cnt1 = cnt1 + plsc.all_reduce_population_count(m1)
                return cnt0, cnt1

            def chunk_s0(off, cnt0):
                d2 = dists(off)
                gidx = lane + (pbase + off)
                m0 = d2 <= r0sq
                c0 = plsc.cumsum(jnp.where(m0, 1, 0))
                p0 = cnt0 + c0
                plsc.store_scatter(idx0, [p0], gidx, mask=m0 & (p0 < lim0))
                return cnt0 + plsc.all_reduce_population_count(m0)

            def condA(st):
                sc_i, cnt0, cnt1 = st
                return (sc_i < _NCHUNK // _UNROLL) & jnp.any(cnt1 < lim1 - 1)

            def bodyA(st):
                sc_i, cnt0, cnt1 = st
                base_off = sc_i * (_UNROLL * _L)
                cnt0, cnt1 = plsc.parallel_loop(
                    base_off, base_off + _UNROLL * _L, _L, unroll=_UNROLL,
                    carry=(cnt0, cnt1))(
                        lambda off, c: chunk_both(off, c[0], c[1]))
                return sc_i + 1, cnt0, cnt1

            sc_i, cnt0, cnt1 = lax.while_loop(
                condA, bodyA, (jnp.int32(0), cnt0_init, cnt1_init))

            def condB(st):
                sc_i, cnt0 = st
                return (sc_i < _NCHUNK // _UNROLL) & jnp.any(cnt0 < lim0 - 1)

            def bodyB(st):
                sc_i, cnt0 = st
                base_off = sc_i * (_UNROLL * _L)
                cnt0 = plsc.parallel_loop(
                    base_off, base_off + _UNROLL * _L, _L, unroll=_UNROLL,
                    carry=cnt0)(chunk_s0)
                return sc_i + 1, cnt0

            _, cnt0 = lax.while_loop(condB, bodyB, (sc_i, cnt0))

            laneo0 = lane + o0
            laneo1 = lane + o1
            first0 = plsc.load_gather(idx0, [zeros_i + o0])
            cur0 = idx0[pl.ds(o0, _L)]
            sel0 = jnp.where(laneo0 - 1 < cnt0, cur0, first0)
            idx0[pl.ds(o0, _L)] = sel0
            loc0 = (sel0 - pbase) * 3
            row0 = laneo0
            plsc.store_scatter(gx0, [row0, zeros_i], plsc.load_gather(pf, [loc0]) - qxb)
            plsc.store_scatter(gx0, [row0, zeros_i + 1], plsc.load_gather(pf, [loc0 + 1]) - qyb)
            plsc.store_scatter(gx0, [row0, zeros_i + 2], plsc.load_gather(pf, [loc0 + 2]) - qzb)
            plsc.store_scatter(gx0, [row0, zeros_i + 3], zeros_f)

            first1 = plsc.load_gather(idx1, [zeros_i + o1])
            for h in range(ns1 // _L):
                cur = idx1[pl.ds(o1 + h * _L, _L)]
                sel = jnp.where(laneo1 + (h * _L - 1) < cnt1, cur, first1)
                idx1[pl.ds(o1 + h * _L, _L)] = sel
                loc = (sel - pbase) * 3
                row = laneo1 + h * _L
                plsc.store_scatter(gx1, [row, zeros_i], plsc.load_gather(pf, [loc]) - qxb)
                plsc.store_scatter(gx1, [row, zeros_i + 1], plsc.load_gather(pf, [loc + 1]) - qyb)
                plsc.store_scatter(gx1, [row, zeros_i + 2], plsc.load_gather(pf, [loc + 2]) - qzb)
                plsc.store_scatter(gx1, [row, zeros_i + 3], zeros_f)
            return carry

        lax.fori_loop(0, _QPW, per_query, 0)

        pltpu.sync_copy(gx0, gx0_h.at[pl.ds(w * n0, n0)])
        pltpu.sync_copy(gx1, gx1_h.at[pl.ds(w * n1, n1)])

        for half in range(2):
            hb = half * 1024
            waits = []
            for cs in range(0, 1024, 128):
                waits.append(pltpu.async_copy(
                    feat_h.at[idx1.at[pl.ds(hb + cs, 128)]],
                    rows.at[pl.ds(cs, 128)], sem))
            for hh in waits:
                hh.wait()
            pltpu.sync_copy(rows, gf1_h.at[pl.ds(w * n1 + hb, 1024)])

        waits = []
        for cs in range(0, n0, 128):
            waits.append(pltpu.async_copy(
                feat_h.at[idx0.at[pl.ds(cs, 128)]], rows.at[pl.ds(cs, 128)], sem))
        for hh in waits:
            hh.wait()
        pltpu.sync_copy(rows, gf0_h.at[pl.ds(w * n0, n0)])

    return k(xyz.reshape(-1), new_xyz.reshape(-1), features)


def _tc_mlp_scale(gx4, gf, layer1, layer2, ns):
    MN = gf.shape[0]
    C1 = layer1["W"].shape[0]
    C2 = layer2["W"].shape[0]
    R = 8192
    NB = MN // R
    qpb = R // ns
    Mq = MN // ns
    cntf = float(MN)

    w1aT = jnp.zeros((4, C1), jnp.float32).at[:3].set(
        layer1["W"][:, :3].T).astype(jnp.bfloat16)
    w1bT = layer1["W"][:, 3:].T.astype(jnp.bfloat16)
    w2T = layer2["W"].T.astype(jnp.bfloat16)
    g1 = layer1["g"].reshape(1, C1)
    b1 = layer1["b"].reshape(1, C1)
    g2 = layer2["g"].reshape(1, C2)
    b2 = layer2["b"].reshape(1, C2)

    def body(gx_ref, gf_ref, w1a_ref, w1b_ref, g1_ref, b1_ref, w2_ref,
             z_ref, st2_ref, st1):
        p = pl.program_id(0)
        i = pl.program_id(1)

        @pl.when((p == 0) & (i == 0))
        def _init():
            st1[...] = jnp.zeros_like(st1)
            st2_ref[...] = jnp.zeros_like(st2_ref)

        y1 = (jnp.dot(gf_ref[...].astype(jnp.bfloat16), w1b_ref[...],
                      preferred_element_type=jnp.float32)
              + jnp.dot(gx_ref[...].astype(jnp.bfloat16), w1a_ref[...],
                        preferred_element_type=jnp.float32))

        @pl.when(p == 0)
        def _pass0():
            st1[0:1, :C1] += jnp.sum(y1, axis=0, keepdims=True)
            st1[1:2, :C1] += jnp.sum(y1 * y1, axis=0, keepdims=True)

        @pl.when(p == 1)
        def _pass1():
            mean1 = st1[0:1, :C1] / cntf
            var1 = st1[1:2, :C1] / cntf - mean1 * mean1
            x = jnp.maximum((y1 - mean1) * (lax.rsqrt(var1 + _EPS) * g1_ref[...])
                            + b1_ref[...], 0.0)
            y2 = jnp.dot(x.astype(jnp.bfloat16), w2_ref[...],
                         preferred_element_type=jnp.float32)
            st2_ref[0:1, :C2] += jnp.sum(y2, axis=0, keepdims=True)
            st2_ref[1:2, :C2] += jnp.sum(y2 * y2, axis=0, keepdims=True)
            z_ref[...] = jnp.max(y2.reshape(qpb, ns, C2), axis=1)

    z, st2 = pl.pallas_call(
        body,
        grid=(2, NB),
        in_specs=[
            pl.BlockSpec((R, 4), lambda p, i: (i, 0)),
            pl.BlockSpec((R, _CIN), lambda p, i: (i, 0)),
            pl.BlockSpec((4, C1), lambda p, i: (0, 0)),
            pl.BlockSpec((_CIN, C1), lambda p, i: (0, 0)),
            pl.BlockSpec((1, C1), lambda p, i: (0, 0)),
            pl.BlockSpec((1, C1), lambda p, i: (0, 0)),
            pl.BlockSpec((C1, C2), lambda p, i: (0, 0)),
        ],
        out_specs=[
            pl.BlockSpec((qpb, C2), lambda p, i: (i, 0)),
            pl.BlockSpec((8, 128), lambda p, i: (0, 0)),
        ],
        out_shape=[
            jax.ShapeDtypeStruct((Mq, C2), jnp.float32),
            jax.ShapeDtypeStruct((8, 128), jnp.float32),
        ],
        scratch_shapes=[pltpu.VMEM((8, 128), jnp.float32)],
    )(gx4, gf, w1aT, w1bT, g1, b1, w2T)

    def fin(z_ref, st_ref, g2_ref, b2_ref, out_ref):
        mean = st_ref[0:1, :C2] / cntf
        var = st_ref[1:2, :C2] / cntf - mean * mean
        out_ref[...] = jnp.maximum(
            (z_ref[...] - mean) * (lax.rsqrt(var + _EPS) * g2_ref[...]) + b2_ref[...],
            0.0)

    out = pl.pallas_call(
        fin,
        out_shape=jax.ShapeDtypeStruct((Mq, C2), jnp.float32),
    )(z, st2, g2, b2)
    return out


def kernel(xyz, xyz_batch_cnt, new_xyz, new_xyz_batch_cnt, features, params):
    gx0, gx1, gf0, gf1 = _sc_ballquery_gather(xyz, new_xyz, features)
    out0 = _tc_mlp_scale(gx0, gf0, params[0][0], params[0][1], _NS[0])
    out1 = _tc_mlp_scale(gx1, gf1, params[1][0], params[1][1], _NS[1])
    new_features = jnp.concatenate([out0, out1], axis=1)
    return new_xyz, new_features


# speedup vs baseline: 80.0280x; 1.3334x over previous
import functools

import jax
import jax.numpy as jnp
from jax import lax
from jax.experimental import pallas as pl
from jax.experimental.pallas import tpu as pltpu
from jax.experimental.pallas import tpu_sc as plsc

_RADII = (0.8, 1.6)
_NS = (16, 32)
_B = 2
_NPTS = 8192
_MQ = 1024
_N = _B * _NPTS
_M = _B * _MQ
_CIN = 32
_C1 = 32
_EPS = 1e-3
_L = 16
_NW = 32
_QPW = _M // _NW
_NCHUNK = _NPTS // _L
_UNROLL = 8


def _sc_ballquery_gather(xyz_flat, new_flat, table0, table1):
    ns0, ns1 = _NS
    r0sq = _RADII[0] * _RADII[0]
    r1sq = _RADII[1] * _RADII[1]
    n0 = _QPW * ns0
    n1 = _QPW * ns1

    mesh = plsc.VectorSubcoreMesh(core_axis_name="c", subcore_axis_name="s")

    out_type = (
        jax.ShapeDtypeStruct((_M * ns0, _C1), jnp.float32),
        jax.ShapeDtypeStruct((_M * ns1, _C1), jnp.float32),
    )
    scratch_types = [
        pltpu.VMEM((_NPTS * 3,), jnp.float32),
        pltpu.VMEM((_NPTS,), jnp.float32),
        pltpu.VMEM((_NPTS,), jnp.float32),
        pltpu.VMEM((_NPTS,), jnp.float32),
        pltpu.VMEM((_QPW * 3,), jnp.float32),
        pltpu.VMEM((n0,), jnp.int32),
        pltpu.VMEM((n1,), jnp.int32),
        pltpu.VMEM((n1, _C1), jnp.float32),
        pltpu.SemaphoreType.DMA,
    ]

    @functools.partial(pl.kernel, out_type=out_type, mesh=mesh,
                       scratch_types=scratch_types,
                       compiler_params=pltpu.CompilerParams(
                           needs_layout_passes=False,
                           use_tc_tiling_on_sc=False))
    def k(xyz_h, new_h, tab0_h, tab1_h, g0_h, g1_h,
          pf, xs, ys, zs, qf, idx0, idx1, rows, sem):
        w = lax.axis_index("c") * 16 + lax.axis_index("s")
        b = w // (_NW // _B)
        pbase = b * _NPTS
        qbase = w * _QPW

        pltpu.sync_copy(xyz_h.at[pl.ds(pbase * 3, _NPTS * 3)], pf)
        pltpu.sync_copy(new_h.at[pl.ds(qbase * 3, _QPW * 3)], qf)

        lane = lax.iota(jnp.int32, _L)
        lane3 = lane * 3
        zeros_i = jnp.zeros((_L,), jnp.int32)

        def soa(it, _):
            base = it * _L
            src = lane3 + base * 3
            xs[pl.ds(base, _L)] = plsc.load_gather(pf, [src])
            ys[pl.ds(base, _L)] = plsc.load_gather(pf, [src + 1])
            zs[pl.ds(base, _L)] = plsc.load_gather(pf, [src + 2])
            return 0
        plsc.parallel_loop(0, _NCHUNK, 1, unroll=8, carry=jnp.int32(0))(soa)

        def per_query(i, carry):
            q3 = zeros_i + i * 3
            qxb = plsc.load_gather(qf, [q3])
            qyb = plsc.load_gather(qf, [q3 + 1])
            qzb = plsc.load_gather(qf, [q3 + 2])
            o0 = i * ns0
            o1 = i * ns1
            cnt0_init = zeros_i + (o0 - 1)
            cnt1_init = zeros_i + (o1 - 1)
            lim0 = zeros_i + (o0 + ns0)
            lim1 = zeros_i + (o1 + ns1)

            def dists(off):
                dx = xs[pl.ds(off, _L)] - qxb
                dy = ys[pl.ds(off, _L)] - qyb
                dz = zs[pl.ds(off, _L)] - qzb
                return dx * dx + dy * dy + dz * dz

            def chunk_both(off, cnt0, cnt1):
                d2 = dists(off)
                gidx = lane + (pbase + off)
                m0 = d2 <= r0sq
                m1 = d2 <= r1sq
                c0 = plsc.cumsum(jnp.where(m0, 1, 0))
                c1 = plsc.cumsum(jnp.where(m1, 1, 0))
                p0 = cnt0 + c0
                p1 = cnt1 + c1
                plsc.store_scatter(idx0, [p0], gidx, mask=m0 & (p0 < lim0))
                plsc.store_scatter(idx1, [p1], gidx, mask=m1 & (p1 < lim1))
                cnt0 = cnt0 + plsc.all_reduce_population_count(m0)
                cnt1 = cnt1 + plsc.all_reduce_population_count(m1)
                return cnt0, cnt1

            def chunk_s0(off, cnt0):
                d2 = dists(off)
                gidx = lane + (pbase + off)
                m0 = d2 <= r0sq
                c0 = plsc.cumsum(jnp.where(m0, 1, 0))
                p0 = cnt0 + c0
                plsc.store_scatter(idx0, [p0], gidx, mask=m0 & (p0 < lim0))
                return cnt0 + plsc.all_reduce_population_count(m0)

            def condA(st):
                sc_i, cnt0, cnt1 = st
                return (sc_i < _NCHUNK // _UNROLL) & jnp.any(cnt1 < lim1 - 1)

            def bodyA(st):
                sc_i, cnt0, cnt1 = st
                base_off = sc_i * (_UNROLL * _L)
                cnt0, cnt1 = plsc.parallel_loop(
                    base_off, base_off + _UNROLL * _L, _L, unroll=_UNROLL,
                    carry=(cnt0, cnt1))(
                        lambda off, c: chunk_both(off, c[0], c[1]))
                return sc_i + 1, cnt0, cnt1

            sc_i, cnt0, cnt1 = lax.while_loop(
                condA, bodyA, (jnp.int32(0), cnt0_init, cnt1_init))

            def condB(st):
                sc_j, cnt0 = st
                return (sc_j < _NCHUNK // _UNROLL) & jnp.any(cnt0 < lim0 - 1)

            def bodyB(st):
                sc_j, cnt0 = st
                base_off = sc_j * (_UNROLL * _L)
                cnt0 = plsc.parallel_loop(
                    base_off, base_off + _UNROLL * _L, _L, unroll=_UNROLL,
                    carry=cnt0)(chunk_s0)
                return sc_j + 1, cnt0

            _, cnt0 = lax.while_loop(condB, bodyB, (sc_i, cnt0))

            laneo0 = lane + o0
            laneo1 = lane + o1
            first0 = plsc.load_gather(idx0, [zeros_i + o0])
            cur0 = idx0[pl.ds(o0, _L)]
            idx0[pl.ds(o0, _L)] = jnp.where(laneo0 - 1 < cnt0, cur0, first0)

            first1 = plsc.load_gather(idx1, [zeros_i + o1])
            for h in range(ns1 // _L):
                cur = idx1[pl.ds(o1 + h * _L, _L)]
                sel = jnp.where(laneo1 + (h * _L - 1) < cnt1, cur, first1)
                idx1[pl.ds(o1 + h * _L, _L)] = sel
            return carry

        lax.fori_loop(0, _QPW, per_query, 0)

        for half in range(2):
            hb = half * 1024
            waits = []
            for cs in range(0, 1024, 128):
                waits.append(pltpu.async_copy(
                    tab1_h.at[idx1.at[pl.ds(hb + cs, 128)]],
                    rows.at[pl.ds(cs, 128)], sem))
            for hh in waits:
                hh.wait()
            pltpu.sync_copy(rows.at[pl.ds(0, 1024)],
                            g1_h.at[pl.ds(w * n1 + hb, 1024)])

        waits = []
        for cs in range(0, n0, 128):
            waits.append(pltpu.async_copy(
                tab0_h.at[idx0.at[pl.ds(cs, 128)]], rows.at[pl.ds(cs, 128)], sem))
        for hh in waits:
            hh.wait()
        pltpu.sync_copy(rows.at[pl.ds(0, n0)], g0_h.at[pl.ds(w * n0, n0)])

    return k(xyz_flat, new_flat, table0, table1)


def _tc_tables(xyz, new_xyz, features, layer1):
    w1aT = layer1["W"][:, :3].T
    w1bT = layer1["W"][:, 3:].T

    def body(xyz_ref, new_ref, feat_ref, w1a_ref, w1b_ref, g_ref, h_ref):
        g_ref[...] = (
            jnp.dot(feat_ref[...], w1b_ref[...], preferred_element_type=jnp.float32)
            + jnp.dot(xyz_ref[...], w1a_ref[...], preferred_element_type=jnp.float32))
        h_ref[...] = jnp.dot(new_ref[...], w1a_ref[...],
                             preferred_element_type=jnp.float32)

    return pl.pallas_call(
        body,
        out_shape=[
            jax.ShapeDtypeStruct((_N, _C1), jnp.float32),
            jax.ShapeDtypeStruct((_M, _C1), jnp.float32),
        ],
    )(xyz, new_xyz, features, w1aT, w1bT)


def _tc_mlp_scale(gpacked, H, layer1, layer2, ns):
    R4 = gpacked.shape[0]
    MN = R4 * 4
    C2 = layer2["W"].shape[0]
    R = 8192
    RB4 = R // 4
    NB = MN // R
    qpb = R // ns
    ns4 = ns // 4
    Mq = MN // ns
    cntf = float(MN)

    w2T = layer2["W"].T
    w2p = jnp.zeros((128, 4 * C2), jnp.float32)
    for g in range(4):
        w2p = w2p.at[g * _C1:(g + 1) * _C1, g * C2:(g + 1) * C2].set(w2T)
    w2p = w2p.astype(jnp.bfloat16)
    g1t = jnp.tile(layer1["g"].reshape(1, _C1), (1, 4))
    b1t = jnp.tile(layer1["b"].reshape(1, _C1), (1, 4))
    g2 = layer2["g"].reshape(1, C2)
    b2 = layer2["b"].reshape(1, C2)

    def body(g_ref, h_ref, g1_ref, b1_ref, w2_ref, z_ref, st2_ref, st1):
        p = pl.program_id(0)
        i = pl.program_id(1)

        @pl.when((p == 0) & (i == 0))
        def _init():
            st1[...] = jnp.zeros_like(st1)
            st2_ref[...] = jnp.zeros_like(st2_ref)

        Hq = h_ref[...]
        Ht = jnp.concatenate([Hq] * 4, axis=1)
        Hexp = jnp.broadcast_to(Ht[:, None, :], (qpb, ns4, 128)).reshape(RB4, 128)
        y1 = g_ref[...] - Hexp

        @pl.when(p == 0)
        def _pass0():
            st1[0:1, :] += jnp.sum(y1, axis=0, keepdims=True)
            st1[1:2, :] += jnp.sum(y1 * y1, axis=0, keepdims=True)

        @pl.when(p == 1)
        def _pass1():
            s1 = st1[0:1, :]
            q1 = st1[1:2, :]
            s1f = (s1[:, 0:32] + s1[:, 32:64]) + (s1[:, 64:96] + s1[:, 96:128])
            q1f = (q1[:, 0:32] + q1[:, 32:64]) + (q1[:, 64:96] + q1[:, 96:128])
            mean1 = s1f / cntf
            var1 = q1f / cntf - mean1 * mean1
            sc = lax.rsqrt(var1 + _EPS)
            mean1t = jnp.concatenate([mean1] * 4, axis=1)
            sct = jnp.concatenate([sc] * 4, axis=1) * g1_ref[...]
            x = jnp.maximum((y1 - mean1t) * sct + b1_ref[...], 0.0)
            y2 = jnp.dot(x.astype(jnp.bfloat16), w2_ref[...],
                         preferred_element_type=jnp.float32)
            s2 = jnp.sum(y2, axis=0, keepdims=True)
            q2 = jnp.sum(y2 * y2, axis=0, keepdims=True)
            s2f = ((s2[:, 0:C2] + s2[:, C2:2 * C2])
                   + (s2[:, 2 * C2:3 * C2] + s2[:, 3 * C2:4 * C2]))
            q2f = ((q2[:, 0:C2] + q2[:, C2:2 * C2])
                   + (q2[:, 2 * C2:3 * C2] + q2[:, 3 * C2:4 * C2]))
            st2_ref[0:1, :C2] += s2f
            st2_ref[1:2, :C2] += q2f
            m4 = jnp.maximum(jnp.maximum(y2[:, 0:C2], y2[:, C2:2 * C2]),
                             jnp.maximum(y2[:, 2 * C2:3 * C2], y2[:, 3 * C2:4 * C2]))
            z_ref[...] = jnp.max(m4.reshape(qpb, ns4, C2), axis=1)

    z, st2 = pl.pallas_call(
        body,
        grid=(2, NB),
        in_specs=[
            pl.BlockSpec((RB4, 128), lambda p, i: (i, 0)),
            pl.BlockSpec((qpb, _C1), lambda p, i: (i, 0)),
            pl.BlockSpec((1, 128), lambda p, i: (0, 0)),
            pl.BlockSpec((1, 128), lambda p, i: (0, 0)),
            pl.BlockSpec((128, 4 * C2), lambda p, i: (0, 0)),
        ],
        out_specs=[
            pl.BlockSpec((qpb, C2), lambda p, i: (i, 0)),
            pl.BlockSpec((8, 128), lambda p, i: (0, 0)),
        ],
        out_shape=[
            jax.ShapeDtypeStruct((Mq, C2), jnp.float32),
            jax.ShapeDtypeStruct((8, 128), jnp.float32),
        ],
        scratch_shapes=[pltpu.VMEM((8, 128), jnp.float32)],
    )(gpacked, H, g1t, b1t, w2p)

    def fin(z_ref, st_ref, g2_ref, b2_ref, out_ref):
        mean = st_ref[0:1, :C2] / cntf
        var = st_ref[1:2, :C2] / cntf - mean * mean
        out_ref[...] = jnp.maximum(
            (z_ref[...] - mean) * (lax.rsqrt(var + _EPS) * g2_ref[...]) + b2_ref[...],
            0.0)

    out = pl.pallas_call(
        fin,
        out_shape=jax.ShapeDtypeStruct((Mq, C2), jnp.float32),
    )(z, st2, g2, b2)
    return out


def kernel(xyz, xyz_batch_cnt, new_xyz, new_xyz_batch_cnt, features, params):
    G0, H0 = _tc_tables(xyz, new_xyz, features, params[0][0])
    G1, H1 = _tc_tables(xyz, new_xyz, features, params[1][0])
    g0, g1 = _sc_ballquery_gather(xyz.reshape(-1), new_xyz.reshape(-1), G0, G1)
    out0 = _tc_mlp_scale(g0.reshape(_M * _NS[0] // 4, 128), H0,
                         params[0][0], params[0][1], _NS[0])
    out1 = _tc_mlp_scale(g1.reshape(_M * _NS[1] // 4, 128), H1,
                         params[1][0], params[1][1], _NS[1])
    new_features = jnp.concatenate([out0, out1], axis=1)
    return new_xyz, new_features


# speedup vs baseline: 84.4093x; 1.0547x over previous
import functools

import jax
import jax.numpy as jnp
from jax import lax
from jax.experimental import pallas as pl
from jax.experimental.pallas import tpu as pltpu
from jax.experimental.pallas import tpu_sc as plsc

_RADII = (0.8, 1.6)
_NS = (16, 32)
_B = 2
_NPTS = 8192
_MQ = 1024
_N = _B * _NPTS
_M = _B * _MQ
_CIN = 32
_C1 = 32
_EPS = 1e-3
_L = 16
_NW = 32
_QPW = _M // _NW
_NCHUNK = _NPTS // _L
_UNROLL = 16


def _sc_ballquery_gather(xyz_flat, new_flat, table0, table1):
    ns0, ns1 = _NS
    r0sq = _RADII[0] * _RADII[0]
    r1sq = _RADII[1] * _RADII[1]
    n0 = _QPW * ns0
    n1 = _QPW * ns1

    mesh = plsc.VectorSubcoreMesh(core_axis_name="c", subcore_axis_name="s")

    out_type = (
        jax.ShapeDtypeStruct((_M * ns0, _C1), jnp.float32),
        jax.ShapeDtypeStruct((_M * ns1, _C1), jnp.float32),
    )
    scratch_types = [
        pltpu.VMEM((_NPTS * 3,), jnp.float32),
        pltpu.VMEM((_NPTS,), jnp.float32),
        pltpu.VMEM((_NPTS,), jnp.float32),
        pltpu.VMEM((_NPTS,), jnp.float32),
        pltpu.VMEM((_QPW * 3,), jnp.float32),
        pltpu.VMEM((n0,), jnp.int32),
        pltpu.VMEM((n1,), jnp.int32),
        pltpu.VMEM((n1, _C1), jnp.float32),
        pltpu.SemaphoreType.DMA,
    ]

    @functools.partial(pl.kernel, out_type=out_type, mesh=mesh,
                       scratch_types=scratch_types,
                       compiler_params=pltpu.CompilerParams(
                           needs_layout_passes=False,
                           use_tc_tiling_on_sc=False))
    def k(xyz_h, new_h, tab0_h, tab1_h, g0_h, g1_h,
          pf, xs, ys, zs, qf, idx0, idx1, rows, sem):
        w = lax.axis_index("c") * 16 + lax.axis_index("s")
        b = w // (_NW // _B)
        pbase = b * _NPTS
        qbase = w * _QPW

        pltpu.sync_copy(xyz_h.at[pl.ds(pbase * 3, _NPTS * 3)], pf)
        pltpu.sync_copy(new_h.at[pl.ds(qbase * 3, _QPW * 3)], qf)

        lane = lax.iota(jnp.int32, _L)
        lane3 = lane * 3
        zeros_i = jnp.zeros((_L,), jnp.int32)

        def soa(it, _):
            base = it * _L
            src = lane3 + base * 3
            xs[pl.ds(base, _L)] = plsc.load_gather(pf, [src])
            ys[pl.ds(base, _L)] = plsc.load_gather(pf, [src + 1])
            zs[pl.ds(base, _L)] = plsc.load_gather(pf, [src + 2])
            return 0
        plsc.parallel_loop(0, _NCHUNK, 1, unroll=8, carry=jnp.int32(0))(soa)

        def per_query(i, carry):
            q3 = zeros_i + i * 3
            qxb = plsc.load_gather(qf, [q3])
            qyb = plsc.load_gather(qf, [q3 + 1])
            qzb = plsc.load_gather(qf, [q3 + 2])
            o0 = i * ns0
            o1 = i * ns1
            cnt0_init = zeros_i + (o0 - 1)
            cnt1_init = zeros_i + (o1 - 1)
            lim0 = zeros_i + (o0 + ns0)
            lim1 = zeros_i + (o1 + ns1)

            def dists(off):
                dx = xs[pl.ds(off, _L)] - qxb
                dy = ys[pl.ds(off, _L)] - qyb
                dz = zs[pl.ds(off, _L)] - qzb
                return dx * dx + dy * dy + dz * dz

            def chunk_both(off, cnt0, cnt1):
                d2 = dists(off)
                gidx = lane + (pbase + off)
                m0 = d2 <= r0sq
                m1 = d2 <= r1sq
                c0 = plsc.cumsum(jnp.where(m0, 1, 0))
                c1 = plsc.cumsum(jnp.where(m1, 1, 0))
                p0 = cnt0 + c0
                p1 = cnt1 + c1
                plsc.store_scatter(idx0, [p0], gidx, mask=m0 & (p0 < lim0))
                plsc.store_scatter(idx1, [p1], gidx, mask=m1 & (p1 < lim1))
                cnt0 = cnt0 + plsc.all_reduce_population_count(m0)
                cnt1 = cnt1 + plsc.all_reduce_population_count(m1)
                return cnt0, cnt1

            def chunk_s0(off, cnt0):
                d2 = dists(off)
                gidx = lane + (pbase + off)
                m0 = d2 <= r0sq
                c0 = plsc.cumsum(jnp.where(m0, 1, 0))
                p0 = cnt0 + c0
                plsc.store_scatter(idx0, [p0], gidx, mask=m0 & (p0 < lim0))
                return cnt0 + plsc.all_reduce_population_count(m0)

            def condA(st):
                sc_i, cnt0, cnt1 = st
                return (sc_i < _NCHUNK // _UNROLL) & jnp.any(cnt1 < lim1 - 1)

            def bodyA(st):
                sc_i, cnt0, cnt1 = st
                base_off = sc_i * (_UNROLL * _L)
                cnt0, cnt1 = plsc.parallel_loop(
                    base_off, base_off + _UNROLL * _L, _L, unroll=_UNROLL,
                    carry=(cnt0, cnt1))(
                        lambda off, c: chunk_both(off, c[0], c[1]))
                return sc_i + 1, cnt0, cnt1

            sc_i, cnt0, cnt1 = lax.while_loop(
                condA, bodyA, (jnp.int32(0), cnt0_init, cnt1_init))

            def condB(st):
                sc_j, cnt0 = st
                return (sc_j < _NCHUNK // _UNROLL) & jnp.any(cnt0 < lim0 - 1)

            def bodyB(st):
                sc_j, cnt0 = st
                base_off = sc_j * (_UNROLL * _L)
                cnt0 = plsc.parallel_loop(
                    base_off, base_off + _UNROLL * _L, _L, unroll=_UNROLL,
                    carry=cnt0)(chunk_s0)
                return sc_j + 1, cnt0

            _, cnt0 = lax.while_loop(condB, bodyB, (sc_i, cnt0))

            laneo0 = lane + o0
            laneo1 = lane + o1
            first0 = plsc.load_gather(idx0, [zeros_i + o0])
            cur0 = idx0[pl.ds(o0, _L)]
            idx0[pl.ds(o0, _L)] = jnp.where(laneo0 - 1 < cnt0, cur0, first0)

            first1 = plsc.load_gather(idx1, [zeros_i + o1])
            for h in range(ns1 // _L):
                cur = idx1[pl.ds(o1 + h * _L, _L)]
                sel = jnp.where(laneo1 + (h * _L - 1) < cnt1, cur, first1)
                idx1[pl.ds(o1 + h * _L, _L)] = sel
            return carry

        lax.fori_loop(0, _QPW, per_query, 0)

        for half in range(2):
            hb = half * 1024
            waits = []
            for cs in range(0, 1024, 128):
                waits.append(pltpu.async_copy(
                    tab1_h.at[idx1.at[pl.ds(hb + cs, 128)]],
                    rows.at[pl.ds(cs, 128)], sem))
            for hh in waits:
                hh.wait()
            pltpu.sync_copy(rows.at[pl.ds(0, 1024)],
                            g1_h.at[pl.ds(w * n1 + hb, 1024)])

        waits = []
        for cs in range(0, n0, 128):
            waits.append(pltpu.async_copy(
                tab0_h.at[idx0.at[pl.ds(cs, 128)]], rows.at[pl.ds(cs, 128)], sem))
        for hh in waits:
            hh.wait()
        pltpu.sync_copy(rows.at[pl.ds(0, n0)], g0_h.at[pl.ds(w * n0, n0)])

    return k(xyz_flat, new_flat, table0, table1)


def _tc_tables(xyz, new_xyz, features, layer1):
    w1aT = layer1["W"][:, :3].T
    w1bT = layer1["W"][:, 3:].T

    def body(xyz_ref, new_ref, feat_ref, w1a_ref, w1b_ref, g_ref, h_ref):
        g_ref[...] = (
            jnp.dot(feat_ref[...], w1b_ref[...], preferred_element_type=jnp.float32)
            + jnp.dot(xyz_ref[...], w1a_ref[...], preferred_element_type=jnp.float32))
        h_ref[...] = jnp.dot(new_ref[...], w1a_ref[...],
                             preferred_element_type=jnp.float32)

    return pl.pallas_call(
        body,
        out_shape=[
            jax.ShapeDtypeStruct((_N, _C1), jnp.float32),
            jax.ShapeDtypeStruct((_M, _C1), jnp.float32),
        ],
    )(xyz, new_xyz, features, w1aT, w1bT)


def _tc_mlp_scale(gpacked, H, layer1, layer2, ns):
    R4 = gpacked.shape[0]
    MN = R4 * 4
    C2 = layer2["W"].shape[0]
    R = 8192
    RB4 = R // 4
    NB = MN // R
    qpb = R // ns
    ns4 = ns // 4
    Mq = MN // ns
    cntf = float(MN)

    w2T = layer2["W"].T
    w2p = jnp.zeros((128, 4 * C2), jnp.float32)
    for g in range(4):
        w2p = w2p.at[g * _C1:(g + 1) * _C1, g * C2:(g + 1) * C2].set(w2T)
    w2p = w2p.astype(jnp.bfloat16)
    g1t = jnp.tile(layer1["g"].reshape(1, _C1), (1, 4))
    b1t = jnp.tile(layer1["b"].reshape(1, _C1), (1, 4))
    g2 = layer2["g"].reshape(1, C2)
    b2 = layer2["b"].reshape(1, C2)

    def body(g_ref, h_ref, g1_ref, b1_ref, w2_ref, z_ref, st2_ref, st1):
        p = pl.program_id(0)
        i = pl.program_id(1)

        @pl.when((p == 0) & (i == 0))
        def _init():
            st1[...] = jnp.zeros_like(st1)
            st2_ref[...] = jnp.zeros_like(st2_ref)

        Hq = h_ref[...]
        Ht = jnp.concatenate([Hq] * 4, axis=1)
        Hexp = jnp.broadcast_to(Ht[:, None, :], (qpb, ns4, 128)).reshape(RB4, 128)
        y1 = g_ref[...] - Hexp

        @pl.when(p == 0)
        def _pass0():
            st1[0:1, :] += jnp.sum(y1, axis=0, keepdims=True)
            st1[1:2, :] += jnp.sum(y1 * y1, axis=0, keepdims=True)

        @pl.when(p == 1)
        def _pass1():
            s1 = st1[0:1, :]
            q1 = st1[1:2, :]
            s1f = (s1[:, 0:32] + s1[:, 32:64]) + (s1[:, 64:96] + s1[:, 96:128])
            q1f = (q1[:, 0:32] + q1[:, 32:64]) + (q1[:, 64:96] + q1[:, 96:128])
            mean1 = s1f / cntf
            var1 = q1f / cntf - mean1 * mean1
            sc = lax.rsqrt(var1 + _EPS)
            mean1t = jnp.concatenate([mean1] * 4, axis=1)
            sct = jnp.concatenate([sc] * 4, axis=1) * g1_ref[...]
            x = jnp.maximum((y1 - mean1t) * sct + b1_ref[...], 0.0)
            y2 = jnp.dot(x.astype(jnp.bfloat16), w2_ref[...],
                         preferred_element_type=jnp.float32)
            s2 = jnp.sum(y2, axis=0, keepdims=True)
            q2 = jnp.sum(y2 * y2, axis=0, keepdims=True)
            s2f = ((s2[:, 0:C2] + s2[:, C2:2 * C2])
                   + (s2[:, 2 * C2:3 * C2] + s2[:, 3 * C2:4 * C2]))
            q2f = ((q2[:, 0:C2] + q2[:, C2:2 * C2])
                   + (q2[:, 2 * C2:3 * C2] + q2[:, 3 * C2:4 * C2]))
            st2_ref[0:1, :C2] += s2f
            st2_ref[1:2, :C2] += q2f
            m4 = jnp.maximum(jnp.maximum(y2[:, 0:C2], y2[:, C2:2 * C2]),
                             jnp.maximum(y2[:, 2 * C2:3 * C2], y2[:, 3 * C2:4 * C2]))
            z_ref[...] = jnp.max(m4.reshape(qpb, ns4, C2), axis=1)

    z, st2 = pl.pallas_call(
        body,
        grid=(2, NB),
        in_specs=[
            pl.BlockSpec((RB4, 128), lambda p, i: (i, 0)),
            pl.BlockSpec((qpb, _C1), lambda p, i: (i, 0)),
            pl.BlockSpec((1, 128), lambda p, i: (0, 0)),
            pl.BlockSpec((1, 128), lambda p, i: (0, 0)),
            pl.BlockSpec((128, 4 * C2), lambda p, i: (0, 0)),
        ],
        out_specs=[
            pl.BlockSpec((qpb, C2), lambda p, i: (i, 0)),
            pl.BlockSpec((8, 128), lambda p, i: (0, 0)),
        ],
        out_shape=[
            jax.ShapeDtypeStruct((Mq, C2), jnp.float32),
            jax.ShapeDtypeStruct((8, 128), jnp.float32),
        ],
        scratch_shapes=[pltpu.VMEM((8, 128), jnp.float32)],
    )(gpacked, H, g1t, b1t, w2p)

    def fin(z_ref, st_ref, g2_ref, b2_ref, out_ref):
        mean = st_ref[0:1, :C2] / cntf
        var = st_ref[1:2, :C2] / cntf - mean * mean
        out_ref[...] = jnp.maximum(
            (z_ref[...] - mean) * (lax.rsqrt(var + _EPS) * g2_ref[...]) + b2_ref[...],
            0.0)

    out = pl.pallas_call(
        fin,
        out_shape=jax.ShapeDtypeStruct((Mq, C2), jnp.float32),
    )(z, st2, g2, b2)
    return out


def kernel(xyz, xyz_batch_cnt, new_xyz, new_xyz_batch_cnt, features, params):
    G0, H0 = _tc_tables(xyz, new_xyz, features, params[0][0])
    G1, H1 = _tc_tables(xyz, new_xyz, features, params[1][0])
    g0, g1 = _sc_ballquery_gather(xyz.reshape(-1), new_xyz.reshape(-1), G0, G1)
    out0 = _tc_mlp_scale(g0.reshape(_M * _NS[0] // 4, 128), H0,
                         params[0][0], params[0][1], _NS[0])
    out1 = _tc_mlp_scale(g1.reshape(_M * _NS[1] // 4, 128), H1,
                         params[1][0], params[1][1], _NS[1])
    new_features = jnp.concatenate([out0, out1], axis=1)
    return new_xyz, new_features


# speedup vs baseline: 95.6559x; 1.1332x over previous
import functools

import jax
import jax.numpy as jnp
from jax import lax
from jax.experimental import pallas as pl
from jax.experimental.pallas import tpu as pltpu
from jax.experimental.pallas import tpu_sc as plsc

_RADII = (0.8, 1.6)
_NS = (16, 32)
_B = 2
_NPTS = 8192
_MQ = 1024
_N = _B * _NPTS
_M = _B * _MQ
_CIN = 32
_C1 = 32
_EPS = 1e-3
_L = 16
_NW = 32
_QPW = _M // _NW
_NCHUNK = _NPTS // _L
_UNROLL = 16


def _sc_ballquery(xyz_flat, new_flat):
    ns0, ns1 = _NS
    r0sq = _RADII[0] * _RADII[0]
    r1sq = _RADII[1] * _RADII[1]
    n0 = _QPW * ns0
    n1 = _QPW * ns1

    mesh = plsc.VectorSubcoreMesh(core_axis_name="c", subcore_axis_name="s")

    out_type = (
        jax.ShapeDtypeStruct((_M * ns0,), jnp.int32),
        jax.ShapeDtypeStruct((_M * ns1,), jnp.int32),
    )
    scratch_types = [
        pltpu.VMEM((_NPTS * 3,), jnp.float32),
        pltpu.VMEM((_NPTS,), jnp.float32),
        pltpu.VMEM((_NPTS,), jnp.float32),
        pltpu.VMEM((_NPTS,), jnp.float32),
        pltpu.VMEM((_QPW * 3,), jnp.float32),
        pltpu.VMEM((n0,), jnp.int32),
        pltpu.VMEM((n1,), jnp.int32),
    ]

    @functools.partial(pl.kernel, out_type=out_type, mesh=mesh,
                       scratch_types=scratch_types,
                       compiler_params=pltpu.CompilerParams(
                           needs_layout_passes=False,
                           use_tc_tiling_on_sc=False))
    def k(xyz_h, new_h, idx0_h, idx1_h,
          pf, xs, ys, zs, qf, idx0, idx1):
        w = lax.axis_index("c") * 16 + lax.axis_index("s")
        b = w // (_NW // _B)
        pbase = b * _NPTS
        qbase = w * _QPW

        pltpu.sync_copy(xyz_h.at[pl.ds(pbase * 3, _NPTS * 3)], pf)
        pltpu.sync_copy(new_h.at[pl.ds(qbase * 3, _QPW * 3)], qf)

        lane = lax.iota(jnp.int32, _L)
        lane3 = lane * 3
        zeros_i = jnp.zeros((_L,), jnp.int32)

        def soa(it, _):
            base = it * _L
            src = lane3 + base * 3
            xs[pl.ds(base, _L)] = plsc.load_gather(pf, [src])
            ys[pl.ds(base, _L)] = plsc.load_gather(pf, [src + 1])
            zs[pl.ds(base, _L)] = plsc.load_gather(pf, [src + 2])
            return 0
        plsc.parallel_loop(0, _NCHUNK, 1, unroll=8, carry=jnp.int32(0))(soa)

        def per_query(i, carry):
            q3 = zeros_i + i * 3
            qxb = plsc.load_gather(qf, [q3])
            qyb = plsc.load_gather(qf, [q3 + 1])
            qzb = plsc.load_gather(qf, [q3 + 2])
            o0 = i * ns0
            o1 = i * ns1
            cnt0_init = zeros_i + (o0 - 1)
            cnt1_init = zeros_i + (o1 - 1)
            lim0 = zeros_i + (o0 + ns0)
            lim1 = zeros_i + (o1 + ns1)

            def dists(off):
                dx = xs[pl.ds(off, _L)] - qxb
                dy = ys[pl.ds(off, _L)] - qyb
                dz = zs[pl.ds(off, _L)] - qzb
                return dx * dx + dy * dy + dz * dz

            def chunk_both(off, cnt0, cnt1):
                d2 = dists(off)
                gidx = lane + (pbase + off)
                m0 = d2 <= r0sq
                m1 = d2 <= r1sq
                c0 = plsc.cumsum(jnp.where(m0, 1, 0))
                c1 = plsc.cumsum(jnp.where(m1, 1, 0))
                p0 = cnt0 + c0
                p1 = cnt1 + c1
                plsc.store_scatter(idx0, [p0], gidx, mask=m0 & (p0 < lim0))
                plsc.store_scatter(idx1, [p1], gidx, mask=m1 & (p1 < lim1))
                cnt0 = cnt0 + plsc.all_reduce_population_count(m0)
                cnt1 = cnt1 + plsc.all_reduce_population_count(m1)
                return cnt0, cnt1

            def chunk_s0(off, cnt0):
                d2 = dists(off)
                gidx = lane + (pbase + off)
                m0 = d2 <= r0sq
                c0 = plsc.cumsum(jnp.where(m0, 1, 0))
                p0 = cnt0 + c0
                plsc.store_scatter(idx0, [p0], gidx, mask=m0 & (p0 < lim0))
                return cnt0 + plsc.all_reduce_population_count(m0)

            def condA(st):
                sc_i, cnt0, cnt1 = st
                return (sc_i < _NCHUNK // _UNROLL) & jnp.any(cnt1 < lim1 - 1)

            def bodyA(st):
                sc_i, cnt0, cnt1 = st
                base_off = sc_i * (_UNROLL * _L)
                cnt0, cnt1 = plsc.parallel_loop(
                    base_off, base_off + _UNROLL * _L, _L, unroll=_UNROLL,
                    carry=(cnt0, cnt1))(
                        lambda off, c: chunk_both(off, c[0], c[1]))
                return sc_i + 1, cnt0, cnt1

            sc_i, cnt0, cnt1 = lax.while_loop(
                condA, bodyA, (jnp.int32(0), cnt0_init, cnt1_init))

            def condB(st):
                sc_j, cnt0 = st
                return (sc_j < _NCHUNK // _UNROLL) & jnp.any(cnt0 < lim0 - 1)

            def bodyB(st):
                sc_j, cnt0 = st
                base_off = sc_j * (_UNROLL * _L)
                cnt0 = plsc.parallel_loop(
                    base_off, base_off + _UNROLL * _L, _L, unroll=_UNROLL,
                    carry=cnt0)(chunk_s0)
                return sc_j + 1, cnt0

            _, cnt0 = lax.while_loop(condB, bodyB, (sc_i, cnt0))

            laneo0 = lane + o0
            laneo1 = lane + o1
            first0 = plsc.load_gather(idx0, [zeros_i + o0])
            cur0 = idx0[pl.ds(o0, _L)]
            idx0[pl.ds(o0, _L)] = jnp.where(laneo0 - 1 < cnt0, cur0, first0)

            first1 = plsc.load_gather(idx1, [zeros_i + o1])
            for h in range(ns1 // _L):
                cur = idx1[pl.ds(o1 + h * _L, _L)]
                sel = jnp.where(laneo1 + (h * _L - 1) < cnt1, cur, first1)
                idx1[pl.ds(o1 + h * _L, _L)] = sel
            return carry

        lax.fori_loop(0, _QPW, per_query, 0)

        pltpu.sync_copy(idx0, idx0_h.at[pl.ds(w * n0, n0)])
        pltpu.sync_copy(idx1, idx1_h.at[pl.ds(w * n1, n1)])

    return k(xyz_flat, new_flat)


def _sc_gather(idx0_all, idx1_all, table0, table1):
    ns0, ns1 = _NS
    n0 = _QPW * ns0
    n1 = _QPW * ns1
    mesh = plsc.VectorSubcoreMesh(core_axis_name="c", subcore_axis_name="s")

    out_type = (
        jax.ShapeDtypeStruct((_M * ns0, _C1), jnp.float32),
        jax.ShapeDtypeStruct((_M * ns1, _C1), jnp.float32),
    )
    scratch_types = [
        pltpu.VMEM((n0,), jnp.int32),
        pltpu.VMEM((n1,), jnp.int32),
        pltpu.VMEM((n1, _C1), jnp.float32),
        pltpu.SemaphoreType.DMA,
    ]

    @functools.partial(pl.kernel, out_type=out_type, mesh=mesh,
                       scratch_types=scratch_types,
                       compiler_params=pltpu.CompilerParams(
                           needs_layout_passes=False,
                           use_tc_tiling_on_sc=False))
    def k(idx0_h, idx1_h, tab0_h, tab1_h, g0_h, g1_h, idx0, idx1, rows, sem):
        w = lax.axis_index("c") * 16 + lax.axis_index("s")
        pltpu.sync_copy(idx0_h.at[pl.ds(w * n0, n0)], idx0)
        pltpu.sync_copy(idx1_h.at[pl.ds(w * n1, n1)], idx1)

        waits = []
        for cs in range(0, n1, 128):
            waits.append(pltpu.async_copy(
                tab1_h.at[idx1.at[pl.ds(cs, 128)]], rows.at[pl.ds(cs, 128)], sem))
        for hh in waits:
            hh.wait()
        pltpu.sync_copy(rows, g1_h.at[pl.ds(w * n1, n1)])

        waits = []
        for cs in range(0, n0, 128):
            waits.append(pltpu.async_copy(
                tab0_h.at[idx0.at[pl.ds(cs, 128)]], rows.at[pl.ds(cs, 128)], sem))
        for hh in waits:
            hh.wait()
        pltpu.sync_copy(rows.at[pl.ds(0, n0)], g0_h.at[pl.ds(w * n0, n0)])

    return k(idx0_all, idx1_all, table0, table1)


def _tc_tables(xyz, new_xyz, features, layer1_0, layer1_1):
    w1aT0 = layer1_0["W"][:, :3].T
    w1bT0 = layer1_0["W"][:, 3:].T
    w1aT1 = layer1_1["W"][:, :3].T
    w1bT1 = layer1_1["W"][:, 3:].T

    def body(xyz_ref, new_ref, feat_ref, a0_ref, b0_ref, a1_ref, b1_ref,
             g0_ref, g1_ref, h0_ref, h1_ref):
        xv = xyz_ref[...]
        nv = new_ref[...]
        fv = feat_ref[...]
        g0_ref[...] = (
            jnp.dot(fv, b0_ref[...], preferred_element_type=jnp.float32)
            + jnp.dot(xv, a0_ref[...], preferred_element_type=jnp.float32))
        g1_ref[...] = (
            jnp.dot(fv, b1_ref[...], preferred_element_type=jnp.float32)
            + jnp.dot(xv, a1_ref[...], preferred_element_type=jnp.float32))
        h0_ref[...] = jnp.dot(nv, a0_ref[...], preferred_element_type=jnp.float32)
        h1_ref[...] = jnp.dot(nv, a1_ref[...], preferred_element_type=jnp.float32)

    return pl.pallas_call(
        body,
        out_shape=[
            jax.ShapeDtypeStruct((_N, _C1), jnp.float32),
            jax.ShapeDtypeStruct((_N, _C1), jnp.float32),
            jax.ShapeDtypeStruct((_M, _C1), jnp.float32),
            jax.ShapeDtypeStruct((_M, _C1), jnp.float32),
        ],
    )(xyz, new_xyz, features, w1aT0, w1bT0, w1aT1, w1bT1)


def _tc_mlp_scale(gpacked, H, layer1, layer2, ns):
    R4 = gpacked.shape[0]
    MN = R4 * 4
    C2 = layer2["W"].shape[0]
    R = 8192
    RB4 = R // 4
    NB = MN // R
    qpb = R // ns
    ns4 = ns // 4
    Mq = MN // ns
    cntf = float(MN)

    w2T = layer2["W"].T
    w2p = jnp.zeros((128, 4 * C2), jnp.float32)
    for g in range(4):
        w2p = w2p.at[g * _C1:(g + 1) * _C1, g * C2:(g + 1) * C2].set(w2T)
    w2p = w2p.astype(jnp.bfloat16)
    g1t = jnp.tile(layer1["g"].reshape(1, _C1), (1, 4))
    b1t = jnp.tile(layer1["b"].reshape(1, _C1), (1, 4))
    g2 = layer2["g"].reshape(1, C2)
    b2 = layer2["b"].reshape(1, C2)

    def body(g_ref, h_ref, g1_ref, b1_ref, w2_ref, z_ref, st2_ref, st1):
        p = pl.program_id(0)
        i = pl.program_id(1)

        @pl.when((p == 0) & (i == 0))
        def _init():
            st1[...] = jnp.zeros_like(st1)
            st2_ref[...] = jnp.zeros_like(st2_ref)

        Hq = h_ref[...]
        Ht = jnp.concatenate([Hq] * 4, axis=1)
        Hexp = jnp.broadcast_to(Ht[:, None, :], (qpb, ns4, 128)).reshape(RB4, 128)
        y1 = g_ref[...] - Hexp

        @pl.when(p == 0)
        def _pass0():
            st1[0:1, :] += jnp.sum(y1, axis=0, keepdims=True)
            st1[1:2, :] += jnp.sum(y1 * y1, axis=0, keepdims=True)

        @pl.when(p == 1)
        def _pass1():
            s1 = st1[0:1, :]
            q1 = st1[1:2, :]
            s1f = (s1[:, 0:32] + s1[:, 32:64]) + (s1[:, 64:96] + s1[:, 96:128])
            q1f = (q1[:, 0:32] + q1[:, 32:64]) + (q1[:, 64:96] + q1[:, 96:128])
            mean1 = s1f / cntf
            var1 = q1f / cntf - mean1 * mean1
            sc = lax.rsqrt(var1 + _EPS)
            mean1t = jnp.concatenate([mean1] * 4, axis=1)
            sct = jnp.concatenate([sc] * 4, axis=1) * g1_ref[...]
            x = jnp.maximum((y1 - mean1t) * sct + b1_ref[...], 0.0)
            y2 = jnp.dot(x.astype(jnp.bfloat16), w2_ref[...],
                         preferred_element_type=jnp.float32)
            s2 = jnp.sum(y2, axis=0, keepdims=True)
            q2 = jnp.sum(y2 * y2, axis=0, keepdims=True)
            s2f = ((s2[:, 0:C2] + s2[:, C2:2 * C2])
                   + (s2[:, 2 * C2:3 * C2] + s2[:, 3 * C2:4 * C2]))
            q2f = ((q2[:, 0:C2] + q2[:, C2:2 * C2])
                   + (q2[:, 2 * C2:3 * C2] + q2[:, 3 * C2:4 * C2]))
            st2_ref[0:1, :C2] += s2f
            st2_ref[1:2, :C2] += q2f
            m4 = jnp.maximum(jnp.maximum(y2[:, 0:C2], y2[:, C2:2 * C2]),
                             jnp.maximum(y2[:, 2 * C2:3 * C2], y2[:, 3 * C2:4 * C2]))
            z_ref[...] = jnp.max(m4.reshape(qpb, ns4, C2), axis=1)

    z, st2 = pl.pallas_call(
        body,
        grid=(2, NB),
        in_specs=[
            pl.BlockSpec((RB4, 128), lambda p, i: (i, 0)),
            pl.BlockSpec((qpb, _C1), lambda p, i: (i, 0)),
            pl.BlockSpec((1, 128), lambda p, i: (0, 0)),
            pl.BlockSpec((1, 128), lambda p, i: (0, 0)),
            pl.BlockSpec((128, 4 * C2), lambda p, i: (0, 0)),
        ],
        out_specs=[
            pl.BlockSpec((qpb, C2), lambda p, i: (i, 0)),
            pl.BlockSpec((8, 128), lambda p, i: (0, 0)),
        ],
        out_shape=[
            jax.ShapeDtypeStruct((Mq, C2), jnp.float32),
            jax.ShapeDtypeStruct((8, 128), jnp.float32),
        ],
        scratch_shapes=[pltpu.VMEM((8, 128), jnp.float32)],
    )(gpacked, H, g1t, b1t, w2p)

    def fin(z_ref, st_ref, g2_ref, b2_ref, out_ref):
        mean = st_ref[0:1, :C2] / cntf
        var = st_ref[1:2, :C2] / cntf - mean * mean
        out_ref[...] = jnp.maximum(
            (z_ref[...] - mean) * (lax.rsqrt(var + _EPS) * g2_ref[...]) + b2_ref[...],
            0.0)

    out = pl.pallas_call(
        fin,
        out_shape=jax.ShapeDtypeStruct((Mq, C2), jnp.float32),
    )(z, st2, g2, b2)
    return out


def kernel(xyz, xyz_batch_cnt, new_xyz, new_xyz_batch_cnt, features, params):
    idx0, idx1 = _sc_ballquery(xyz.reshape(-1), new_xyz.reshape(-1))
    G0, G1, H0, H1 = _tc_tables(xyz, new_xyz, features,
                                params[0][0], params[1][0])
    g0, g1 = _sc_gather(idx0, idx1, G0, G1)
    out0 = _tc_mlp_scale(g0.reshape(_M * _NS[0] // 4, 128), H0,
                         params[0][0], params[0][1], _NS[0])
    out1 = _tc_mlp_scale(g1.reshape(_M * _NS[1] // 4, 128), H1,
                         params[1][0], params[1][1], _NS[1])
    new_features = jnp.concatenate([out0, out1], axis=1)
    return new_xyz, new_features
